# Initial kernel scaffold; baseline (speedup 1.0000x reference)
import functools

import jax
import jax.numpy as jnp
from jax import lax
from jax.experimental import pallas as pl
from jax.experimental.pallas import tpu as pltpu
from jax.experimental.pallas import tpu_sc as plsc

NC = 2
NS = 16
LANES = 16
EPS = 1e-5
ROW_BLK = 1000


def _vector_mesh():
    return plsc.VectorSubcoreMesh(
        core_axis_name="c", subcore_axis_name="s", num_cores=NC, num_subcores=NS
    )


def _sc_hist(srcm, dstm, srcr, dstr, n):
    e = srcm.shape[0]
    ec = e // (NC * NS)

    @functools.partial(
        pl.kernel,
        out_type=jax.ShapeDtypeStruct((NC * NS, 4 * n), jnp.float32),
        mesh=_vector_mesh(),
        scratch_types=[
            pltpu.VMEM((4 * n,), jnp.float32),
            pltpu.VMEM((ec,), jnp.int32),
        ],
    )
    def hist_kernel(srcm_hbm, dstm_hbm, srcr_hbm, dstr_hbm, out_hbm, hist_v, idx_v):
        c = lax.axis_index("c")
        s = lax.axis_index("s")
        wid = c * NS + s

        zeros16 = jnp.zeros((LANES,), jnp.float32)

        @pl.loop(0, 4 * n, step=LANES)
        def _(i):
            hist_v[pl.ds(i, LANES)] = zeros16

        ones16 = jnp.ones((LANES,), jnp.float32)
        base = wid * ec
        for k, ref in enumerate((srcm_hbm, dstm_hbm, srcr_hbm, dstr_hbm)):
            pltpu.sync_copy(ref.at[pl.ds(base, ec)], idx_v)
            kvec = jnp.full((LANES,), k, jnp.int32)

            @pl.loop(0, ec, step=LANES)
            def _(i):
                idx = idx_v[pl.ds(i, LANES)]
                plsc.addupdate_scatter(hist_v, [idx * 4 + kvec], ones16)

        pltpu.sync_copy(hist_v, out_hbm.at[wid])

    return hist_kernel(srcm, dstm, srcr, dstr)


def _sc_scatter(hs, hr, srcm, dstm, srcr, dstr, zrows):
    n, h = hs.shape
    e = srcm.shape[0]
    blk = 128
    nblk = e // blk
    per = nblk // NS
    rem = nblk % NS
    rpt = n // NS

    out_sds = jax.ShapeDtypeStruct((n, h), jnp.float32)

    @functools.partial(
        pl.kernel,
        out_type=(out_sds, out_sds),
        mesh=_vector_mesh(),
        scratch_types=[
            pltpu.VMEM_SHARED((n, h), jnp.float32),
            pltpu.VMEM((rpt, h), jnp.float32),
            pltpu.VMEM((blk, h), jnp.float32),
            pltpu.VMEM((blk,), jnp.int32),
            pltpu.VMEM((blk,), jnp.int32),
            pltpu.SemaphoreType.DMA,
        ],
    )
    def scatter_kernel(hs_hbm, hr_hbm, srcm_hbm, dstm_hbm, srcr_hbm, dstr_hbm,
                       zrows_hbm, aggs_hbm, aggr_hbm,
                       acc_sh, zbuf_v, rows_v, idxs_v, idxd_v, sem):
        c = lax.axis_index("c")
        s = lax.axis_index("s")

        pltpu.sync_copy(zrows_hbm, zbuf_v)
        pltpu.sync_copy(zbuf_v, acc_sh.at[pl.ds(s * rpt, rpt)])
        plsc.subcore_barrier()

        nb = per + jnp.where(s < rem, 1, 0)

        def run_graph(h_hbm, src_hbm, dst_hbm):
            def body(i, carry):
                base = (s + NS * i) * blk
                pltpu.sync_copy(src_hbm.at[pl.ds(base, blk)], idxs_v)
                pltpu.sync_copy(dst_hbm.at[pl.ds(base, blk)], idxd_v)
                pltpu.async_copy(h_hbm.at[idxs_v], rows_v, sem).wait()
                pltpu.sync_copy(rows_v, acc_sh.at[idxd_v], add=True)
                return carry

            lax.fori_loop(0, nb, body, 0)

        @pl.when(c == 0)
        def _():
            run_graph(hs_hbm, srcm_hbm, dstm_hbm)

        @pl.when(c == 1)
        def _():
            run_graph(hr_hbm, srcr_hbm, dstr_hbm)

        plsc.subcore_barrier()

        @pl.when(c == 0)
        def _():
            pltpu.sync_copy(acc_sh.at[pl.ds(s * rpt, rpt)],
                            aggs_hbm.at[pl.ds(s * rpt, rpt)])

        @pl.when(c == 1)
        def _():
            pltpu.sync_copy(acc_sh.at[pl.ds(s * rpt, rpt)],
                            aggr_hbm.at[pl.ds(s * rpt, rpt)])

    return scatter_kernel(hs, hr, srcm, dstm, srcr, dstr, zrows)


def _tc_pre(histp, x, sw1, rw1):
    n, d = x.shape
    g = n // ROW_BLK
    nt = NC * NS

    def body(hp_ref, x_ref, sw_ref, rw_ref, h1s_ref, h1r_ref, rsd_ref):
        deg = jnp.sum(hp_ref[...], axis=0)
        rsd = lax.rsqrt(jnp.maximum(deg, 1.0))
        rsd_ref[...] = rsd
        xs = x_ref[...]
        h1s_ref[...] = jnp.dot(xs * rsd[:, 0:1], sw_ref[...],
                               preferred_element_type=jnp.float32)
        h1r_ref[...] = jnp.dot(xs * rsd[:, 2:3], rw_ref[...],
                               preferred_element_type=jnp.float32)

    return pl.pallas_call(
        body,
        grid=(g,),
        in_specs=[
            pl.BlockSpec((nt, ROW_BLK, 4), lambda i: (0, i, 0)),
            pl.BlockSpec((ROW_BLK, d), lambda i: (i, 0)),
            pl.BlockSpec((d, d), lambda i: (0, 0)),
            pl.BlockSpec((d, d), lambda i: (0, 0)),
        ],
        out_specs=[
            pl.BlockSpec((ROW_BLK, d), lambda i: (i, 0)),
            pl.BlockSpec((ROW_BLK, d), lambda i: (i, 0)),
            pl.BlockSpec((ROW_BLK, 4), lambda i: (i, 0)),
        ],
        out_shape=[
            jax.ShapeDtypeStruct((n, d), jnp.float32),
            jax.ShapeDtypeStruct((n, d), jnp.float32),
            jax.ShapeDtypeStruct((n, 4), jnp.float32),
        ],
    )(histp, x, sw1, rw1)


def _elu(v):
    return jnp.where(v > 0, v, jnp.expm1(v))


def _tc_stage_a(aggs, aggr, rsd, bs, br):
    n, d = aggs.shape
    g = n // ROW_BLK

    def body(as_ref, ar_ref, rsd_ref, bs_ref, br_ref, ys_ref, yr_ref, ps_ref):
        rsd = rsd_ref[...]
        ys = _elu(as_ref[...] * rsd[:, 1:2] + bs_ref[...])
        yr = _elu(ar_ref[...] * rsd[:, 3:4] + br_ref[...])
        ys_ref[...] = ys
        yr_ref[...] = yr
        ps_ref[...] = jnp.concatenate(
            [
                jnp.sum(ys, axis=0).reshape(1, 1, d),
                jnp.sum(ys * ys, axis=0).reshape(1, 1, d),
                jnp.sum(yr, axis=0).reshape(1, 1, d),
                jnp.sum(yr * yr, axis=0).reshape(1, 1, d),
            ],
            axis=1,
        )

    return pl.pallas_call(
        body,
        grid=(g,),
        in_specs=[
            pl.BlockSpec((ROW_BLK, d), lambda i: (i, 0)),
            pl.BlockSpec((ROW_BLK, d), lambda i: (i, 0)),
            pl.BlockSpec((ROW_BLK, 4), lambda i: (i, 0)),
            pl.BlockSpec((1, d), lambda i: (0, 0)),
            pl.BlockSpec((1, d), lambda i: (0, 0)),
        ],
        out_specs=[
            pl.BlockSpec((ROW_BLK, d), lambda i: (i, 0)),
            pl.BlockSpec((ROW_BLK, d), lambda i: (i, 0)),
            pl.BlockSpec((1, 4, d), lambda i: (i, 0, 0)),
        ],
        out_shape=[
            jax.ShapeDtypeStruct((n, d), jnp.float32),
            jax.ShapeDtypeStruct((n, d), jnp.float32),
            jax.ShapeDtypeStruct((g, 4, d), jnp.float32),
        ],
    )(aggs, aggr, rsd, bs, br)


def _bn_from_sums(y, tot_s1, tot_s2, n, gamma, beta):
    mu = tot_s1 / n
    var = tot_s2 / n - mu * mu
    return (y - mu) * lax.rsqrt(var + EPS) * gamma + beta


def _tc_mid_b(ys, yr, ps, rsd, gs, bes, gr, ber, sw2, rw2):
    n, d = ys.shape
    g = n // ROW_BLK

    def body(ys_ref, yr_ref, ps_ref, rsd_ref, gs_ref, bes_ref, gr_ref, ber_ref,
             sw_ref, rw_ref, h2s_ref, h2r_ref):
        tot = jnp.sum(ps_ref[...], axis=0)
        rsd = rsd_ref[...]
        zs = _bn_from_sums(ys_ref[...], tot[0:1], tot[1:2], n, gs_ref[...], bes_ref[...])
        zr = _bn_from_sums(yr_ref[...], tot[2:3], tot[3:4], n, gr_ref[...], ber_ref[...])
        h2s_ref[...] = jnp.dot(zs * rsd[:, 0:1], sw_ref[...],
                               preferred_element_type=jnp.float32)
        h2r_ref[...] = jnp.dot(zr * rsd[:, 2:3], rw_ref[...],
                               preferred_element_type=jnp.float32)

    return pl.pallas_call(
        body,
        grid=(g,),
        in_specs=[
            pl.BlockSpec((ROW_BLK, d), lambda i: (i, 0)),
            pl.BlockSpec((ROW_BLK, d), lambda i: (i, 0)),
            pl.BlockSpec((g, 4, d), lambda i: (0, 0, 0)),
            pl.BlockSpec((ROW_BLK, 4), lambda i: (i, 0)),
            pl.BlockSpec((1, d), lambda i: (0, 0)),
            pl.BlockSpec((1, d), lambda i: (0, 0)),
            pl.BlockSpec((1, d), lambda i: (0, 0)),
            pl.BlockSpec((1, d), lambda i: (0, 0)),
            pl.BlockSpec((d, d), lambda i: (0, 0)),
            pl.BlockSpec((d, d), lambda i: (0, 0)),
        ],
        out_specs=[
            pl.BlockSpec((ROW_BLK, d), lambda i: (i, 0)),
            pl.BlockSpec((ROW_BLK, d), lambda i: (i, 0)),
        ],
        out_shape=[
            jax.ShapeDtypeStruct((n, d), jnp.float32),
            jax.ShapeDtypeStruct((n, d), jnp.float32),
        ],
    )(ys, yr, ps, rsd, gs, bes, gr, ber, sw2, rw2)


def _tc_final(ys, yr, ps, gs, bes, gr, ber, g11, g12, g13, g14, gb1, pa, gw2,
              gb2, fw, fb):
    n, d = ys.shape
    c_out = fw.shape[1]
    g = n // ROW_BLK

    def body(ys_ref, yr_ref, ps_ref, gs_ref, bes_ref, gr_ref, ber_ref,
             g11_ref, g12_ref, g13_ref, g14_ref, gb1_ref, pa_ref, gw2_ref,
             gb2_ref, fw_ref, fb_ref, out_ref):
        tot = jnp.sum(ps_ref[...], axis=0)
        h2 = _bn_from_sums(ys_ref[...], tot[0:1], tot[1:2], n, gs_ref[...], bes_ref[...])
        h3 = _bn_from_sums(yr_ref[...], tot[2:3], tot[3:4], n, gr_ref[...], ber_ref[...])
        diff = jnp.abs(h2 - h3)
        prod = h2 * h3
        z = (jnp.dot(h2, g11_ref[...], preferred_element_type=jnp.float32)
             + jnp.dot(h3, g12_ref[...], preferred_element_type=jnp.float32)
             + jnp.dot(diff, g13_ref[...], preferred_element_type=jnp.float32)
             + jnp.dot(prod, g14_ref[...], preferred_element_type=jnp.float32)
             + gb1_ref[...])
        z = jnp.where(z > 0, z, pa_ref[...] * z)
        zz = jnp.dot(z, gw2_ref[...], preferred_element_type=jnp.float32) + gb2_ref[...]
        gate = 1.0 / (1.0 + jnp.exp(-zz))
        comb = gate * h2 + (1.0 - gate) * h3
        out_ref[...] = jnp.dot(comb, fw_ref[...],
                               preferred_element_type=jnp.float32) + fb_ref[...]

    return pl.pallas_call(
        body,
        grid=(g,),
        in_specs=[
            pl.BlockSpec((ROW_BLK, d), lambda i: (i, 0)),
            pl.BlockSpec((ROW_BLK, d), lambda i: (i, 0)),
            pl.BlockSpec((g, 4, d), lambda i: (0, 0, 0)),
            pl.BlockSpec((1, d), lambda i: (0, 0)),
            pl.BlockSpec((1, d), lambda i: (0, 0)),
            pl.BlockSpec((1, d), lambda i: (0, 0)),
            pl.BlockSpec((1, d), lambda i: (0, 0)),
            pl.BlockSpec((d, d), lambda i: (0, 0)),
            pl.BlockSpec((d, d), lambda i: (0, 0)),
            pl.BlockSpec((d, d), lambda i: (0, 0)),
            pl.BlockSpec((d, d), lambda i: (0, 0)),
            pl.BlockSpec((1, d), lambda i: (0, 0)),
            pl.BlockSpec((1, 1), lambda i: (0, 0)),
            pl.BlockSpec((d, d), lambda i: (0, 0)),
            pl.BlockSpec((1, d), lambda i: (0, 0)),
            pl.BlockSpec((d, c_out), lambda i: (0, 0)),
            pl.BlockSpec((1, c_out), lambda i: (0, 0)),
        ],
        out_specs=pl.BlockSpec((ROW_BLK, c_out), lambda i: (i, 0)),
        out_shape=jax.ShapeDtypeStruct((n, c_out), jnp.float32),
    )(ys, yr, ps, gs, bes, gr, ber, g11, g12, g13, g14, gb1, pa, gw2, gb2, fw, fb)


def kernel(node_features, mention_edges, retweet_edges, sW1, sb1, sg1, sbe1,
           sW2, sb2, sg2, sbe2, rW1, rb1, rg1, rbe1, rW2, rb2, rg2, rbe2,
           gW1, gb1, pa, gW2, gb2, fW, fb):
    n, d = node_features.shape
    srcm, dstm = mention_edges[0], mention_edges[1]
    srcr, dstr = retweet_edges[0], retweet_edges[1]

    histp = _sc_hist(srcm, dstm, srcr, dstr, n).reshape(NC * NS, n, 4)
    h1s, h1r, rsd = _tc_pre(histp, node_features, sW1, rW1)

    zrows = jnp.zeros((n // NS, d), jnp.float32)
    agg1s, agg1r = _sc_scatter(h1s, h1r, srcm, dstm, srcr, dstr, zrows)
    ys1, yr1, ps1 = _tc_stage_a(agg1s, agg1r, rsd,
                                sb1.reshape(1, -1), rb1.reshape(1, -1))
    h2s, h2r = _tc_mid_b(ys1, yr1, ps1, rsd,
                         sg1.reshape(1, -1), sbe1.reshape(1, -1),
                         rg1.reshape(1, -1), rbe1.reshape(1, -1), sW2, rW2)

    agg2s, agg2r = _sc_scatter(h2s, h2r, srcm, dstm, srcr, dstr, zrows)
    ys2, yr2, ps2 = _tc_stage_a(agg2s, agg2r, rsd,
                                sb2.reshape(1, -1), rb2.reshape(1, -1))

    out = _tc_final(ys2, yr2, ps2,
                    sg2.reshape(1, -1), sbe2.reshape(1, -1),
                    rg2.reshape(1, -1), rbe2.reshape(1, -1),
                    gW1[0:d], gW1[d:2 * d], gW1[2 * d:3 * d], gW1[3 * d:4 * d],
                    gb1.reshape(1, -1), pa.reshape(1, 1), gW2,
                    gb2.reshape(1, -1), fW, fb.reshape(1, -1))
    return out


# speedup vs baseline: 4.8125x; 4.8125x over previous
import dataclasses
import functools

import jax
import jax.numpy as jnp
from jax import lax
from jax.experimental import pallas as pl
from jax.experimental.pallas import tpu as pltpu
from jax.experimental.pallas import tpu_sc as plsc

NC = 2
NS = 16
LANES = 16
EPS = 1e-5
ROW_BLK = 1000


def _vector_mesh():
    return plsc.VectorSubcoreMesh(
        core_axis_name="c", subcore_axis_name="s", num_cores=NC, num_subcores=NS
    )


def _sc_compiler_params():
    cp = pltpu.CompilerParams()
    if "needs_layout_passes" in pltpu.CompilerParams.__dataclass_fields__:
        cp = dataclasses.replace(cp, needs_layout_passes=False)
    return cp


def _sc_hist(srcm, dstm, srcr, dstr, n):
    e = srcm.shape[0]
    ec = e // (NC * NS)

    @functools.partial(
        pl.kernel,
        out_type=jax.ShapeDtypeStruct((NC * NS, 4 * n), jnp.float32),
        mesh=_vector_mesh(),
        scratch_types=[
            pltpu.VMEM((4 * n,), jnp.float32),
            pltpu.VMEM((ec,), jnp.int32),
        ],
        compiler_params=_sc_compiler_params(),
    )
    def hist_kernel(srcm_hbm, dstm_hbm, srcr_hbm, dstr_hbm, out_hbm, hist_v, idx_v):
        c = lax.axis_index("c")
        s = lax.axis_index("s")
        wid = c * NS + s

        zeros16 = jnp.zeros((LANES,), jnp.float32)

        @pl.loop(0, 4 * n, step=LANES)
        def _(i):
            hist_v[pl.ds(i, LANES)] = zeros16

        ones16 = jnp.ones((LANES,), jnp.float32)
        base = wid * ec
        for k, ref in enumerate((srcm_hbm, dstm_hbm, srcr_hbm, dstr_hbm)):
            pltpu.sync_copy(ref.at[pl.ds(base, ec)], idx_v)
            kvec = jnp.full((LANES,), k, jnp.int32)

            @pl.loop(0, ec, step=LANES)
            def _(i):
                idx = idx_v[pl.ds(i, LANES)]
                plsc.addupdate_scatter(hist_v, [idx * 4 + kvec], ones16)

        pltpu.sync_copy(hist_v, out_hbm.at[wid])

    return hist_kernel(srcm, dstm, srcr, dstr)


def _sc_scatter(hs, hr, srcm, dstm, srcr, dstr, zrows):
    n, h = hs.shape
    e = srcm.shape[0]
    blk = 128
    nblk = e // blk
    per = nblk // NS
    rem = nblk % NS
    rpt = (n // NS) // 8 * 8
    tail = n - NS * rpt

    out_sds = jax.ShapeDtypeStruct((n, h), jnp.float32)

    @functools.partial(
        pl.kernel,
        out_type=(out_sds, out_sds),
        mesh=_vector_mesh(),
        scratch_types=[
            pltpu.VMEM_SHARED((n, h), jnp.float32),
            pltpu.VMEM((blk, h), jnp.float32),
            pltpu.VMEM((blk,), jnp.int32),
            pltpu.VMEM((blk,), jnp.int32),
            pltpu.SemaphoreType.DMA,
        ],
    )
    def scatter_kernel(hs_hbm, hr_hbm, srcm_hbm, dstm_hbm, srcr_hbm, dstr_hbm,
                       zrows_hbm, aggs_hbm, aggr_hbm,
                       acc_sh, rows_v, idxs_v, idxd_v, sem):
        c = lax.axis_index("c")
        s = lax.axis_index("s")

        pltpu.sync_copy(zrows_hbm, rows_v)

        @pl.loop(0, 4)
        def _(i):
            pltpu.sync_copy(rows_v, acc_sh.at[pl.ds(s * rpt + i * blk, blk)])

        pltpu.sync_copy(rows_v.at[pl.ds(0, rpt - 4 * blk)],
                        acc_sh.at[pl.ds(s * rpt + 4 * blk, rpt - 4 * blk)])

        @pl.when(s == NS - 1)
        def _():
            pltpu.sync_copy(rows_v.at[pl.ds(0, tail)],
                            acc_sh.at[pl.ds(NS * rpt, tail)])

        plsc.subcore_barrier()

        nb = per + jnp.where(s < rem, 1, 0)

        def run_graph(h_hbm, src_hbm, dst_hbm):
            def body(i, carry):
                base = (s + NS * i) * blk
                pltpu.sync_copy(src_hbm.at[pl.ds(base, blk)], idxs_v)
                pltpu.sync_copy(dst_hbm.at[pl.ds(base, blk)], idxd_v)
                pltpu.async_copy(h_hbm.at[idxs_v], rows_v, sem).wait()
                pltpu.sync_copy(rows_v, acc_sh.at[idxd_v], add=True)
                return carry

            lax.fori_loop(0, nb, body, 0)

        @pl.when(c == 0)
        def _():
            run_graph(hs_hbm, srcm_hbm, dstm_hbm)

        @pl.when(c == 1)
        def _():
            run_graph(hr_hbm, srcr_hbm, dstr_hbm)

        plsc.subcore_barrier()

        @pl.when(c == 0)
        def _():
            pltpu.sync_copy(acc_sh.at[pl.ds(s * rpt, rpt)],
                            aggs_hbm.at[pl.ds(s * rpt, rpt)])

            @pl.when(s == NS - 1)
            def _():
                pltpu.sync_copy(acc_sh.at[pl.ds(NS * rpt, tail)],
                                aggs_hbm.at[pl.ds(NS * rpt, tail)])

        @pl.when(c == 1)
        def _():
            pltpu.sync_copy(acc_sh.at[pl.ds(s * rpt, rpt)],
                            aggr_hbm.at[pl.ds(s * rpt, rpt)])

            @pl.when(s == NS - 1)
            def _():
                pltpu.sync_copy(acc_sh.at[pl.ds(NS * rpt, tail)],
                                aggr_hbm.at[pl.ds(NS * rpt, tail)])

    return scatter_kernel(hs, hr, srcm, dstm, srcr, dstr, zrows)


def _tc_pre(histp, x, sw1, rw1):
    n, d = x.shape
    g = n // ROW_BLK
    nt = NC * NS

    def body(hp_ref, x_ref, sw_ref, rw_ref, h1s_ref, h1r_ref, rsd_ref):
        deg = jnp.sum(hp_ref[...], axis=0)
        rsd = lax.rsqrt(jnp.maximum(deg, 1.0))
        rsd_ref[...] = rsd
        xs = x_ref[...]
        h1s_ref[...] = jnp.dot(xs * rsd[:, 0:1], sw_ref[...],
                               preferred_element_type=jnp.float32)
        h1r_ref[...] = jnp.dot(xs * rsd[:, 2:3], rw_ref[...],
                               preferred_element_type=jnp.float32)

    return pl.pallas_call(
        body,
        grid=(g,),
        in_specs=[
            pl.BlockSpec((nt, ROW_BLK, 4), lambda i: (0, i, 0)),
            pl.BlockSpec((ROW_BLK, d), lambda i: (i, 0)),
            pl.BlockSpec((d, d), lambda i: (0, 0)),
            pl.BlockSpec((d, d), lambda i: (0, 0)),
        ],
        out_specs=[
            pl.BlockSpec((ROW_BLK, d), lambda i: (i, 0)),
            pl.BlockSpec((ROW_BLK, d), lambda i: (i, 0)),
            pl.BlockSpec((ROW_BLK, 4), lambda i: (i, 0)),
        ],
        out_shape=[
            jax.ShapeDtypeStruct((n, d), jnp.float32),
            jax.ShapeDtypeStruct((n, d), jnp.float32),
            jax.ShapeDtypeStruct((n, 4), jnp.float32),
        ],
    )(histp, x, sw1, rw1)


def _elu(v):
    return jnp.where(v > 0, v, jnp.exp(jnp.minimum(v, 0.0)) - 1.0)


def _tc_stage_a(aggs, aggr, rsd, bs, br):
    n, d = aggs.shape
    g = n // ROW_BLK

    def body(as_ref, ar_ref, rsd_ref, bs_ref, br_ref, ys_ref, yr_ref, ps_ref):
        rsd = rsd_ref[...]
        ys = _elu(as_ref[...] * rsd[:, 1:2] + bs_ref[...])
        yr = _elu(ar_ref[...] * rsd[:, 3:4] + br_ref[...])
        ys_ref[...] = ys
        yr_ref[...] = yr
        ps_ref[...] = jnp.concatenate(
            [
                jnp.sum(ys, axis=0).reshape(1, 1, d),
                jnp.sum(ys * ys, axis=0).reshape(1, 1, d),
                jnp.sum(yr, axis=0).reshape(1, 1, d),
                jnp.sum(yr * yr, axis=0).reshape(1, 1, d),
            ],
            axis=1,
        )

    return pl.pallas_call(
        body,
        grid=(g,),
        in_specs=[
            pl.BlockSpec((ROW_BLK, d), lambda i: (i, 0)),
            pl.BlockSpec((ROW_BLK, d), lambda i: (i, 0)),
            pl.BlockSpec((ROW_BLK, 4), lambda i: (i, 0)),
            pl.BlockSpec((1, d), lambda i: (0, 0)),
            pl.BlockSpec((1, d), lambda i: (0, 0)),
        ],
        out_specs=[
            pl.BlockSpec((ROW_BLK, d), lambda i: (i, 0)),
            pl.BlockSpec((ROW_BLK, d), lambda i: (i, 0)),
            pl.BlockSpec((1, 4, d), lambda i: (i, 0, 0)),
        ],
        out_shape=[
            jax.ShapeDtypeStruct((n, d), jnp.float32),
            jax.ShapeDtypeStruct((n, d), jnp.float32),
            jax.ShapeDtypeStruct((g, 4, d), jnp.float32),
        ],
    )(aggs, aggr, rsd, bs, br)


def _bn_from_sums(y, tot_s1, tot_s2, n, gamma, beta):
    mu = tot_s1 / n
    var = tot_s2 / n - mu * mu
    return (y - mu) * lax.rsqrt(var + EPS) * gamma + beta


def _tc_mid_b(ys, yr, ps, rsd, gs, bes, gr, ber, sw2, rw2):
    n, d = ys.shape
    g = n // ROW_BLK

    def body(ys_ref, yr_ref, ps_ref, rsd_ref, gs_ref, bes_ref, gr_ref, ber_ref,
             sw_ref, rw_ref, h2s_ref, h2r_ref):
        tot = jnp.sum(ps_ref[...], axis=0)
        rsd = rsd_ref[...]
        zs = _bn_from_sums(ys_ref[...], tot[0:1], tot[1:2], n, gs_ref[...], bes_ref[...])
        zr = _bn_from_sums(yr_ref[...], tot[2:3], tot[3:4], n, gr_ref[...], ber_ref[...])
        h2s_ref[...] = jnp.dot(zs * rsd[:, 0:1], sw_ref[...],
                               preferred_element_type=jnp.float32)
        h2r_ref[...] = jnp.dot(zr * rsd[:, 2:3], rw_ref[...],
                               preferred_element_type=jnp.float32)

    return pl.pallas_call(
        body,
        grid=(g,),
        in_specs=[
            pl.BlockSpec((ROW_BLK, d), lambda i: (i, 0)),
            pl.BlockSpec((ROW_BLK, d), lambda i: (i, 0)),
            pl.BlockSpec((g, 4, d), lambda i: (0, 0, 0)),
            pl.BlockSpec((ROW_BLK, 4), lambda i: (i, 0)),
            pl.BlockSpec((1, d), lambda i: (0, 0)),
            pl.BlockSpec((1, d), lambda i: (0, 0)),
            pl.BlockSpec((1, d), lambda i: (0, 0)),
            pl.BlockSpec((1, d), lambda i: (0, 0)),
            pl.BlockSpec((d, d), lambda i: (0, 0)),
            pl.BlockSpec((d, d), lambda i: (0, 0)),
        ],
        out_specs=[
            pl.BlockSpec((ROW_BLK, d), lambda i: (i, 0)),
            pl.BlockSpec((ROW_BLK, d), lambda i: (i, 0)),
        ],
        out_shape=[
            jax.ShapeDtypeStruct((n, d), jnp.float32),
            jax.ShapeDtypeStruct((n, d), jnp.float32),
        ],
    )(ys, yr, ps, rsd, gs, bes, gr, ber, sw2, rw2)


def _tc_final(ys, yr, ps, gs, bes, gr, ber, g11, g12, g13, g14, gb1, pa, gw2,
              gb2, fw, fb):
    n, d = ys.shape
    c_out = fw.shape[1]
    g = n // ROW_BLK

    def body(ys_ref, yr_ref, ps_ref, gs_ref, bes_ref, gr_ref, ber_ref,
             g11_ref, g12_ref, g13_ref, g14_ref, gb1_ref, pa_ref, gw2_ref,
             gb2_ref, fw_ref, fb_ref, out_ref):
        tot = jnp.sum(ps_ref[...], axis=0)
        h2 = _bn_from_sums(ys_ref[...], tot[0:1], tot[1:2], n, gs_ref[...], bes_ref[...])
        h3 = _bn_from_sums(yr_ref[...], tot[2:3], tot[3:4], n, gr_ref[...], ber_ref[...])
        diff = jnp.abs(h2 - h3)
        prod = h2 * h3
        z = (jnp.dot(h2, g11_ref[...], preferred_element_type=jnp.float32)
             + jnp.dot(h3, g12_ref[...], preferred_element_type=jnp.float32)
             + jnp.dot(diff, g13_ref[...], preferred_element_type=jnp.float32)
             + jnp.dot(prod, g14_ref[...], preferred_element_type=jnp.float32)
             + gb1_ref[...])
        z = jnp.where(z > 0, z, pa_ref[...] * z)
        zz = jnp.dot(z, gw2_ref[...], preferred_element_type=jnp.float32) + gb2_ref[...]
        gate = 1.0 / (1.0 + jnp.exp(-zz))
        comb = gate * h2 + (1.0 - gate) * h3
        out_ref[...] = jnp.dot(comb, fw_ref[...],
                               preferred_element_type=jnp.float32) + fb_ref[...]

    return pl.pallas_call(
        body,
        grid=(g,),
        in_specs=[
            pl.BlockSpec((ROW_BLK, d), lambda i: (i, 0)),
            pl.BlockSpec((ROW_BLK, d), lambda i: (i, 0)),
            pl.BlockSpec((g, 4, d), lambda i: (0, 0, 0)),
            pl.BlockSpec((1, d), lambda i: (0, 0)),
            pl.BlockSpec((1, d), lambda i: (0, 0)),
            pl.BlockSpec((1, d), lambda i: (0, 0)),
            pl.BlockSpec((1, d), lambda i: (0, 0)),
            pl.BlockSpec((d, d), lambda i: (0, 0)),
            pl.BlockSpec((d, d), lambda i: (0, 0)),
            pl.BlockSpec((d, d), lambda i: (0, 0)),
            pl.BlockSpec((d, d), lambda i: (0, 0)),
            pl.BlockSpec((1, d), lambda i: (0, 0)),
            pl.BlockSpec((1, 1), lambda i: (0, 0)),
            pl.BlockSpec((d, d), lambda i: (0, 0)),
            pl.BlockSpec((1, d), lambda i: (0, 0)),
            pl.BlockSpec((d, c_out), lambda i: (0, 0)),
            pl.BlockSpec((1, c_out), lambda i: (0, 0)),
        ],
        out_specs=pl.BlockSpec((ROW_BLK, c_out), lambda i: (i, 0)),
        out_shape=jax.ShapeDtypeStruct((n, c_out), jnp.float32),
    )(ys, yr, ps, gs, bes, gr, ber, g11, g12, g13, g14, gb1, pa, gw2, gb2, fw, fb)


def kernel(node_features, mention_edges, retweet_edges, sW1, sb1, sg1, sbe1,
           sW2, sb2, sg2, sbe2, rW1, rb1, rg1, rbe1, rW2, rb2, rg2, rbe2,
           gW1, gb1, pa, gW2, gb2, fW, fb):
    n, d = node_features.shape
    srcm, dstm = mention_edges[0], mention_edges[1]
    srcr, dstr = retweet_edges[0], retweet_edges[1]

    histp = _sc_hist(srcm, dstm, srcr, dstr, n).reshape(NC * NS, n, 4)
    h1s, h1r, rsd = _tc_pre(histp, node_features, sW1, rW1)

    zrows = jnp.zeros((128, d), jnp.float32)
    agg1s, agg1r = _sc_scatter(h1s, h1r, srcm, dstm, srcr, dstr, zrows)
    ys1, yr1, ps1 = _tc_stage_a(agg1s, agg1r, rsd,
                                sb1.reshape(1, -1), rb1.reshape(1, -1))
    h2s, h2r = _tc_mid_b(ys1, yr1, ps1, rsd,
                         sg1.reshape(1, -1), sbe1.reshape(1, -1),
                         rg1.reshape(1, -1), rbe1.reshape(1, -1), sW2, rW2)

    agg2s, agg2r = _sc_scatter(h2s, h2r, srcm, dstm, srcr, dstr, zrows)
    ys2, yr2, ps2 = _tc_stage_a(agg2s, agg2r, rsd,
                                sb2.reshape(1, -1), rb2.reshape(1, -1))

    out = _tc_final(ys2, yr2, ps2,
                    sg2.reshape(1, -1), sbe2.reshape(1, -1),
                    rg2.reshape(1, -1), rbe2.reshape(1, -1),
                    gW1[0:d], gW1[d:2 * d], gW1[2 * d:3 * d], gW1[3 * d:4 * d],
                    gb1.reshape(1, -1), pa.reshape(1, 1), gW2,
                    gb2.reshape(1, -1), fW, fb.reshape(1, -1))
    return out


# speedup vs baseline: 8.4030x; 1.7461x over previous
import dataclasses
import functools

import jax
import jax.numpy as jnp
from jax import lax
from jax.experimental import pallas as pl
from jax.experimental.pallas import tpu as pltpu
from jax.experimental.pallas import tpu_sc as plsc

NC = 2
NS = 16
LANES = 16
EPS = 1e-5
ROW_BLK = 1000


def _vector_mesh():
    return plsc.VectorSubcoreMesh(
        core_axis_name="c", subcore_axis_name="s", num_cores=NC, num_subcores=NS
    )


def _sc_compiler_params():
    cp = pltpu.CompilerParams()
    if "needs_layout_passes" in pltpu.CompilerParams.__dataclass_fields__:
        cp = dataclasses.replace(cp, needs_layout_passes=False)
    return cp


def _sc_hist(srcm, dstm, srcr, dstr, n):
    e = srcm.shape[0]
    ec = e // (NC * NS)

    @functools.partial(
        pl.kernel,
        out_type=jax.ShapeDtypeStruct((NC * NS, 4 * n), jnp.float32),
        mesh=_vector_mesh(),
        scratch_types=[
            pltpu.VMEM((4 * n,), jnp.float32),
            pltpu.VMEM((ec,), jnp.int32),
        ],
        compiler_params=_sc_compiler_params(),
    )
    def hist_kernel(srcm_hbm, dstm_hbm, srcr_hbm, dstr_hbm, out_hbm, hist_v, idx_v):
        c = lax.axis_index("c")
        s = lax.axis_index("s")
        wid = c * NS + s

        zeros16 = jnp.zeros((LANES,), jnp.float32)

        @pl.loop(0, 4 * n, step=LANES)
        def _(i):
            hist_v[pl.ds(i, LANES)] = zeros16

        ones16 = jnp.ones((LANES,), jnp.float32)
        base = wid * ec
        for k, ref in enumerate((srcm_hbm, dstm_hbm, srcr_hbm, dstr_hbm)):
            pltpu.sync_copy(ref.at[pl.ds(base, ec)], idx_v)
            kvec = jnp.full((LANES,), k, jnp.int32)

            @pl.loop(0, ec, step=LANES)
            def _(i):
                idx = idx_v[pl.ds(i, LANES)]
                plsc.addupdate_scatter(hist_v, [idx * 4 + kvec], ones16)

        pltpu.sync_copy(hist_v, out_hbm.at[wid])

    return hist_kernel(srcm, dstm, srcr, dstr)


def _sc_scatter(hs, hr, srcm2, dstm2, srcr2, dstr2, zrows):
    n, h = hs.shape
    blk = 128
    cblk = 10
    nchunk = srcm2.shape[0]
    cper = nchunk // NS
    crem = nchunk % NS
    rpt = (n // NS) // 8 * 8
    tail = n - NS * rpt

    out_sds = jax.ShapeDtypeStruct((n, h), jnp.float32)

    @functools.partial(
        pl.kernel,
        out_type=(out_sds, out_sds),
        mesh=_vector_mesh(),
        scratch_types=[
            pltpu.VMEM_SHARED((n, h), jnp.float32),
            pltpu.VMEM((blk, h), jnp.float32),
            pltpu.VMEM((blk, h), jnp.float32),
            pltpu.VMEM((cblk, blk), jnp.int32),
            pltpu.VMEM((cblk, blk), jnp.int32),
            pltpu.VMEM((cblk, blk), jnp.int32),
            pltpu.VMEM((cblk, blk), jnp.int32),
            pltpu.SemaphoreType.DMA,
            pltpu.SemaphoreType.DMA,
            pltpu.SemaphoreType.DMA,
            pltpu.SemaphoreType.DMA,
            pltpu.SemaphoreType.DMA,
            pltpu.SemaphoreType.DMA,
        ],
    )
    def scatter_kernel(hs_hbm, hr_hbm, srcm_hbm, dstm_hbm, srcr_hbm, dstr_hbm,
                       zrows_hbm, aggs_hbm, aggr_hbm,
                       acc_sh, rows_a, rows_b, isrc_a, isrc_b, idst_a, idst_b,
                       semg_a, semg_b, sems_a, sems_b, semi_a, semi_b):
        c = lax.axis_index("c")
        s = lax.axis_index("s")
        rows = (rows_a, rows_b)
        isrc = (isrc_a, isrc_b)
        idst = (idst_a, idst_b)
        semg = (semg_a, semg_b)
        sems = (sems_a, sems_b)
        semi = (semi_a, semi_b)

        pltpu.sync_copy(zrows_hbm, rows_a)

        @pl.loop(0, 4)
        def _(i):
            pltpu.sync_copy(rows_a, acc_sh.at[pl.ds(s * rpt + i * blk, blk)])

        pltpu.sync_copy(rows_a.at[pl.ds(0, rpt - 4 * blk)],
                        acc_sh.at[pl.ds(s * rpt + 4 * blk, rpt - 4 * blk)])

        @pl.when(s == NS - 1)
        def _():
            pltpu.sync_copy(rows_a.at[pl.ds(0, tail)],
                            acc_sh.at[pl.ds(NS * rpt, tail)])

        plsc.subcore_barrier()

        ccount = cper + jnp.where(s < crem, 1, 0)

        def run_graph(h_hbm, src_hbm, dst_hbm):
            def issue_idx(chunk, par):
                gc = s + NS * chunk
                pltpu.async_copy(src_hbm.at[gc], isrc[par], semi[par])
                pltpu.async_copy(dst_hbm.at[gc], idst[par], semi[par])

            def wait_idx(par):
                pltpu.make_async_copy(src_hbm.at[0], isrc[par],
                                      semi[par]).wait()
                pltpu.make_async_copy(dst_hbm.at[0], idst[par],
                                      semi[par]).wait()

            def wait_rows(sem):
                pltpu.make_async_copy(hs_hbm.at[pl.ds(0, blk)], rows_a,
                                      sem).wait()

            def process_chunk(chunk, par, is_first):
                wait_idx(par)
                for j in range(cblk):
                    b = j & 1
                    if is_first and j < 2:
                        pass
                    else:
                        wait_rows(sems[b])
                    pltpu.async_copy(h_hbm.at[isrc[par].at[j]], rows[b],
                                     semg[b])
                    if j == 1:
                        @pl.when(chunk + 1 < ccount)
                        def _():
                            issue_idx(chunk + 1, 1 - par)
                    if j >= 1:
                        pb = 1 - b
                        wait_rows(semg[pb])
                        pltpu.async_copy(rows[pb],
                                         acc_sh.at[idst[par].at[j - 1]],
                                         sems[pb], add=True)
                last = (cblk - 1) & 1
                wait_rows(semg[last])
                pltpu.async_copy(rows[last],
                                 acc_sh.at[idst[par].at[cblk - 1]],
                                 sems[last], add=True)

            issue_idx(0, 0)
            process_chunk(0, 0, True)

            def pair_body(i, carry):
                process_chunk(1 + 2 * i, 1, False)
                process_chunk(2 + 2 * i, 0, False)
                return carry

            lax.fori_loop(0, (cper - 1) // 2, pair_body, 0)

            @pl.when(s < crem)
            def _():
                process_chunk(cper, 1, False)

            wait_rows(sems[0])
            wait_rows(sems[1])

        @pl.when(c == 0)
        def _():
            run_graph(hs_hbm, srcm_hbm, dstm_hbm)

        @pl.when(c == 1)
        def _():
            run_graph(hr_hbm, srcr_hbm, dstr_hbm)

        plsc.subcore_barrier()

        @pl.when(c == 0)
        def _():
            pltpu.sync_copy(acc_sh.at[pl.ds(s * rpt, rpt)],
                            aggs_hbm.at[pl.ds(s * rpt, rpt)])

            @pl.when(s == NS - 1)
            def _():
                pltpu.sync_copy(acc_sh.at[pl.ds(NS * rpt, tail)],
                                aggs_hbm.at[pl.ds(NS * rpt, tail)])

        @pl.when(c == 1)
        def _():
            pltpu.sync_copy(acc_sh.at[pl.ds(s * rpt, rpt)],
                            aggr_hbm.at[pl.ds(s * rpt, rpt)])

            @pl.when(s == NS - 1)
            def _():
                pltpu.sync_copy(acc_sh.at[pl.ds(NS * rpt, tail)],
                                aggr_hbm.at[pl.ds(NS * rpt, tail)])

    return scatter_kernel(hs, hr, srcm2, dstm2, srcr2, dstr2, zrows)


def _tc_pre(histp, x, sw1, rw1):
    n, d = x.shape
    g = n // ROW_BLK
    nt = NC * NS

    def body(hp_ref, x_ref, sw_ref, rw_ref, h1s_ref, h1r_ref, rsd_ref):
        deg = jnp.sum(hp_ref[...], axis=0)
        rsd = lax.rsqrt(jnp.maximum(deg, 1.0))
        rsd_ref[...] = rsd
        xs = x_ref[...]
        h1s_ref[...] = jnp.dot(xs * rsd[:, 0:1], sw_ref[...],
                               preferred_element_type=jnp.float32)
        h1r_ref[...] = jnp.dot(xs * rsd[:, 2:3], rw_ref[...],
                               preferred_element_type=jnp.float32)

    return pl.pallas_call(
        body,
        grid=(g,),
        in_specs=[
            pl.BlockSpec((nt, ROW_BLK, 4), lambda i: (0, i, 0)),
            pl.BlockSpec((ROW_BLK, d), lambda i: (i, 0)),
            pl.BlockSpec((d, d), lambda i: (0, 0)),
            pl.BlockSpec((d, d), lambda i: (0, 0)),
        ],
        out_specs=[
            pl.BlockSpec((ROW_BLK, d), lambda i: (i, 0)),
            pl.BlockSpec((ROW_BLK, d), lambda i: (i, 0)),
            pl.BlockSpec((ROW_BLK, 4), lambda i: (i, 0)),
        ],
        out_shape=[
            jax.ShapeDtypeStruct((n, d), jnp.float32),
            jax.ShapeDtypeStruct((n, d), jnp.float32),
            jax.ShapeDtypeStruct((n, 4), jnp.float32),
        ],
    )(histp, x, sw1, rw1)


def _elu(v):
    return jnp.where(v > 0, v, jnp.exp(jnp.minimum(v, 0.0)) - 1.0)


def _tc_stage_a(aggs, aggr, rsd, bs, br):
    n, d = aggs.shape
    g = n // ROW_BLK

    def body(as_ref, ar_ref, rsd_ref, bs_ref, br_ref, ys_ref, yr_ref, ps_ref):
        rsd = rsd_ref[...]
        ys = _elu(as_ref[...] * rsd[:, 1:2] + bs_ref[...])
        yr = _elu(ar_ref[...] * rsd[:, 3:4] + br_ref[...])
        ys_ref[...] = ys
        yr_ref[...] = yr
        ps_ref[...] = jnp.concatenate(
            [
                jnp.sum(ys, axis=0).reshape(1, 1, d),
                jnp.sum(ys * ys, axis=0).reshape(1, 1, d),
                jnp.sum(yr, axis=0).reshape(1, 1, d),
                jnp.sum(yr * yr, axis=0).reshape(1, 1, d),
            ],
            axis=1,
        )

    return pl.pallas_call(
        body,
        grid=(g,),
        in_specs=[
            pl.BlockSpec((ROW_BLK, d), lambda i: (i, 0)),
            pl.BlockSpec((ROW_BLK, d), lambda i: (i, 0)),
            pl.BlockSpec((ROW_BLK, 4), lambda i: (i, 0)),
            pl.BlockSpec((1, d), lambda i: (0, 0)),
            pl.BlockSpec((1, d), lambda i: (0, 0)),
        ],
        out_specs=[
            pl.BlockSpec((ROW_BLK, d), lambda i: (i, 0)),
            pl.BlockSpec((ROW_BLK, d), lambda i: (i, 0)),
            pl.BlockSpec((1, 4, d), lambda i: (i, 0, 0)),
        ],
        out_shape=[
            jax.ShapeDtypeStruct((n, d), jnp.float32),
            jax.ShapeDtypeStruct((n, d), jnp.float32),
            jax.ShapeDtypeStruct((g, 4, d), jnp.float32),
        ],
    )(aggs, aggr, rsd, bs, br)


def _bn_from_sums(y, tot_s1, tot_s2, n, gamma, beta):
    mu = tot_s1 / n
    var = tot_s2 / n - mu * mu
    return (y - mu) * lax.rsqrt(var + EPS) * gamma + beta


def _tc_mid_b(ys, yr, ps, rsd, gs, bes, gr, ber, sw2, rw2):
    n, d = ys.shape
    g = n // ROW_BLK

    def body(ys_ref, yr_ref, ps_ref, rsd_ref, gs_ref, bes_ref, gr_ref, ber_ref,
             sw_ref, rw_ref, h2s_ref, h2r_ref):
        tot = jnp.sum(ps_ref[...], axis=0)
        rsd = rsd_ref[...]
        zs = _bn_from_sums(ys_ref[...], tot[0:1], tot[1:2], n, gs_ref[...], bes_ref[...])
        zr = _bn_from_sums(yr_ref[...], tot[2:3], tot[3:4], n, gr_ref[...], ber_ref[...])
        h2s_ref[...] = jnp.dot(zs * rsd[:, 0:1], sw_ref[...],
                               preferred_element_type=jnp.float32)
        h2r_ref[...] = jnp.dot(zr * rsd[:, 2:3], rw_ref[...],
                               preferred_element_type=jnp.float32)

    return pl.pallas_call(
        body,
        grid=(g,),
        in_specs=[
            pl.BlockSpec((ROW_BLK, d), lambda i: (i, 0)),
            pl.BlockSpec((ROW_BLK, d), lambda i: (i, 0)),
            pl.BlockSpec((g, 4, d), lambda i: (0, 0, 0)),
            pl.BlockSpec((ROW_BLK, 4), lambda i: (i, 0)),
            pl.BlockSpec((1, d), lambda i: (0, 0)),
            pl.BlockSpec((1, d), lambda i: (0, 0)),
            pl.BlockSpec((1, d), lambda i: (0, 0)),
            pl.BlockSpec((1, d), lambda i: (0, 0)),
            pl.BlockSpec((d, d), lambda i: (0, 0)),
            pl.BlockSpec((d, d), lambda i: (0, 0)),
        ],
        out_specs=[
            pl.BlockSpec((ROW_BLK, d), lambda i: (i, 0)),
            pl.BlockSpec((ROW_BLK, d), lambda i: (i, 0)),
        ],
        out_shape=[
            jax.ShapeDtypeStruct((n, d), jnp.float32),
            jax.ShapeDtypeStruct((n, d), jnp.float32),
        ],
    )(ys, yr, ps, rsd, gs, bes, gr, ber, sw2, rw2)


def _tc_final(ys, yr, ps, gs, bes, gr, ber, g11, g12, g13, g14, gb1, pa, gw2,
              gb2, fw, fb):
    n, d = ys.shape
    c_out = fw.shape[1]
    g = n // ROW_BLK

    def body(ys_ref, yr_ref, ps_ref, gs_ref, bes_ref, gr_ref, ber_ref,
             g11_ref, g12_ref, g13_ref, g14_ref, gb1_ref, pa_ref, gw2_ref,
             gb2_ref, fw_ref, fb_ref, out_ref):
        tot = jnp.sum(ps_ref[...], axis=0)
        h2 = _bn_from_sums(ys_ref[...], tot[0:1], tot[1:2], n, gs_ref[...], bes_ref[...])
        h3 = _bn_from_sums(yr_ref[...], tot[2:3], tot[3:4], n, gr_ref[...], ber_ref[...])
        diff = jnp.abs(h2 - h3)
        prod = h2 * h3
        z = (jnp.dot(h2, g11_ref[...], preferred_element_type=jnp.float32)
             + jnp.dot(h3, g12_ref[...], preferred_element_type=jnp.float32)
             + jnp.dot(diff, g13_ref[...], preferred_element_type=jnp.float32)
             + jnp.dot(prod, g14_ref[...], preferred_element_type=jnp.float32)
             + gb1_ref[...])
        z = jnp.where(z > 0, z, pa_ref[...] * z)
        zz = jnp.dot(z, gw2_ref[...], preferred_element_type=jnp.float32) + gb2_ref[...]
        gate = 1.0 / (1.0 + jnp.exp(-zz))
        comb = gate * h2 + (1.0 - gate) * h3
        out_ref[...] = jnp.dot(comb, fw_ref[...],
                               preferred_element_type=jnp.float32) + fb_ref[...]

    return pl.pallas_call(
        body,
        grid=(g,),
        in_specs=[
            pl.BlockSpec((ROW_BLK, d), lambda i: (i, 0)),
            pl.BlockSpec((ROW_BLK, d), lambda i: (i, 0)),
            pl.BlockSpec((g, 4, d), lambda i: (0, 0, 0)),
            pl.BlockSpec((1, d), lambda i: (0, 0)),
            pl.BlockSpec((1, d), lambda i: (0, 0)),
            pl.BlockSpec((1, d), lambda i: (0, 0)),
            pl.BlockSpec((1, d), lambda i: (0, 0)),
            pl.BlockSpec((d, d), lambda i: (0, 0)),
            pl.BlockSpec((d, d), lambda i: (0, 0)),
            pl.BlockSpec((d, d), lambda i: (0, 0)),
            pl.BlockSpec((d, d), lambda i: (0, 0)),
            pl.BlockSpec((1, d), lambda i: (0, 0)),
            pl.BlockSpec((1, 1), lambda i: (0, 0)),
            pl.BlockSpec((d, d), lambda i: (0, 0)),
            pl.BlockSpec((1, d), lambda i: (0, 0)),
            pl.BlockSpec((d, c_out), lambda i: (0, 0)),
            pl.BlockSpec((1, c_out), lambda i: (0, 0)),
        ],
        out_specs=pl.BlockSpec((ROW_BLK, c_out), lambda i: (i, 0)),
        out_shape=jax.ShapeDtypeStruct((n, c_out), jnp.float32),
    )(ys, yr, ps, gs, bes, gr, ber, g11, g12, g13, g14, gb1, pa, gw2, gb2, fw, fb)


def kernel(node_features, mention_edges, retweet_edges, sW1, sb1, sg1, sbe1,
           sW2, sb2, sg2, sbe2, rW1, rb1, rg1, rbe1, rW2, rb2, rg2, rbe2,
           gW1, gb1, pa, gW2, gb2, fW, fb):
    n, d = node_features.shape
    srcm, dstm = mention_edges[0], mention_edges[1]
    srcr, dstr = retweet_edges[0], retweet_edges[1]
    srcm2, dstm2 = srcm.reshape(-1, 10, 128), dstm.reshape(-1, 10, 128)
    srcr2, dstr2 = srcr.reshape(-1, 10, 128), dstr.reshape(-1, 10, 128)

    histp = _sc_hist(srcm, dstm, srcr, dstr, n).reshape(NC * NS, n, 4)
    h1s, h1r, rsd = _tc_pre(histp, node_features, sW1, rW1)

    zrows = jnp.zeros((128, d), jnp.float32)
    agg1s, agg1r = _sc_scatter(h1s, h1r, srcm2, dstm2, srcr2, dstr2, zrows)
    ys1, yr1, ps1 = _tc_stage_a(agg1s, agg1r, rsd,
                                sb1.reshape(1, -1), rb1.reshape(1, -1))
    h2s, h2r = _tc_mid_b(ys1, yr1, ps1, rsd,
                         sg1.reshape(1, -1), sbe1.reshape(1, -1),
                         rg1.reshape(1, -1), rbe1.reshape(1, -1), sW2, rW2)

    agg2s, agg2r = _sc_scatter(h2s, h2r, srcm2, dstm2, srcr2, dstr2, zrows)
    ys2, yr2, ps2 = _tc_stage_a(agg2s, agg2r, rsd,
                                sb2.reshape(1, -1), rb2.reshape(1, -1))

    out = _tc_final(ys2, yr2, ps2,
                    sg2.reshape(1, -1), sbe2.reshape(1, -1),
                    rg2.reshape(1, -1), rbe2.reshape(1, -1),
                    gW1[0:d], gW1[d:2 * d], gW1[2 * d:3 * d], gW1[3 * d:4 * d],
                    gb1.reshape(1, -1), pa.reshape(1, 1), gW2,
                    gb2.reshape(1, -1), fW, fb.reshape(1, -1))
    return out


# speedup vs baseline: 8.8415x; 1.0522x over previous
import dataclasses
import functools

import jax
import jax.numpy as jnp
from jax import lax
from jax.experimental import pallas as pl
from jax.experimental.pallas import tpu as pltpu
from jax.experimental.pallas import tpu_sc as plsc

NC = 2
NS = 16
LANES = 16
EPS = 1e-5
ROW_BLK = 1000


def _vector_mesh():
    return plsc.VectorSubcoreMesh(
        core_axis_name="c", subcore_axis_name="s", num_cores=NC, num_subcores=NS
    )


def _sc_compiler_params():
    cp = pltpu.CompilerParams()
    if "needs_layout_passes" in pltpu.CompilerParams.__dataclass_fields__:
        cp = dataclasses.replace(cp, needs_layout_passes=False)
    return cp


def _sc_hist(srcm, dstm, srcr, dstr, n):
    e = srcm.shape[0]
    ec = e // (NC * NS)

    @functools.partial(
        pl.kernel,
        out_type=jax.ShapeDtypeStruct((NC * NS, 4 * n), jnp.float32),
        mesh=_vector_mesh(),
        scratch_types=[
            pltpu.VMEM((4 * n,), jnp.float32),
            pltpu.VMEM((ec,), jnp.int32),
        ],
        compiler_params=_sc_compiler_params(),
    )
    def hist_kernel(srcm_hbm, dstm_hbm, srcr_hbm, dstr_hbm, out_hbm, hist_v, idx_v):
        c = lax.axis_index("c")
        s = lax.axis_index("s")
        wid = c * NS + s

        zeros16 = jnp.zeros((LANES,), jnp.float32)

        @pl.loop(0, 4 * n, step=LANES)
        def _(i):
            hist_v[pl.ds(i, LANES)] = zeros16

        ones16 = jnp.ones((LANES,), jnp.float32)
        base = wid * ec
        for k, ref in enumerate((srcm_hbm, dstm_hbm, srcr_hbm, dstr_hbm)):
            pltpu.sync_copy(ref.at[pl.ds(base, ec)], idx_v)
            kvec = jnp.full((LANES,), k, jnp.int32)

            @pl.loop(0, ec, step=LANES)
            def _(i):
                idx = idx_v[pl.ds(i, LANES)]
                plsc.addupdate_scatter(hist_v, [idx * 4 + kvec], ones16)

        pltpu.sync_copy(hist_v, out_hbm.at[wid])

    return hist_kernel(srcm, dstm, srcr, dstr)


def _sc_scatter(hs, hr, edm, edr, zrows):
    n, h = hs.shape
    blk = 128
    nblk = edm.shape[0]
    bper = nblk // NS
    brem = nblk % NS
    NR = 3
    NI = 5
    main_iters = (bper - 3 - 3) // (NR * NI)
    tail0 = 3 + main_iters * NR * NI
    rpt = (n // NS) // 8 * 8
    tail = n - NS * rpt

    out_sds = jax.ShapeDtypeStruct((n, h), jnp.float32)

    @functools.partial(
        pl.kernel,
        out_type=(out_sds, out_sds),
        mesh=_vector_mesh(),
        scratch_types=(
            [pltpu.VMEM_SHARED((n, h), jnp.float32)]
            + [pltpu.VMEM((blk, h), jnp.float32)] * NR
            + [pltpu.VMEM((2, blk), jnp.int32)] * NI
            + [pltpu.SemaphoreType.DMA] * (2 * NR + NI)
        ),
    )
    def scatter_kernel(hs_hbm, hr_hbm, edm_hbm, edr_hbm,
                       zrows_hbm, aggs_hbm, aggr_hbm,
                       acc_sh, *bufs):
        c = lax.axis_index("c")
        s = lax.axis_index("s")
        rows = bufs[:NR]
        ibuf = bufs[NR:NR + NI]
        semg = bufs[NR + NI:2 * NR + NI]
        sems = bufs[2 * NR + NI:3 * NR + NI]
        semi = bufs[3 * NR + NI:3 * NR + 2 * NI]
        rows_a = rows[0]

        pltpu.sync_copy(zrows_hbm, rows_a)

        @pl.loop(0, 4)
        def _(i):
            pltpu.sync_copy(rows_a, acc_sh.at[pl.ds(s * rpt + i * blk, blk)])

        pltpu.sync_copy(rows_a.at[pl.ds(0, rpt - 4 * blk)],
                        acc_sh.at[pl.ds(s * rpt + 4 * blk, rpt - 4 * blk)])

        @pl.when(s == NS - 1)
        def _():
            pltpu.sync_copy(rows_a.at[pl.ds(0, tail)],
                            acc_sh.at[pl.ds(NS * rpt, tail)])

        plsc.subcore_barrier()

        count = bper + jnp.where(s < brem, 1, 0)

        def run_graph(h_hbm, ed_hbm):
            def issue_idx(t, p):
                pltpu.async_copy(ed_hbm.at[s + NS * t], ibuf[p], semi[p])

            def wait_idx(p):
                pltpu.make_async_copy(ed_hbm.at[0], ibuf[p], semi[p]).wait()

            def wait_rows(sem):
                pltpu.make_async_copy(hs_hbm.at[pl.ds(0, blk)], rows_a,
                                      sem).wait()

            def do_block(t, to, skip_wait, skip_prev):
                rp, ip = to % NR, to % NI
                if not skip_wait:
                    wait_rows(sems[rp])
                wait_idx(ip)
                pltpu.async_copy(h_hbm.at[ibuf[ip].at[0]], rows[rp], semg[rp])

                @pl.when(t + 2 < count)
                def _():
                    issue_idx(t + 2, (to + 2) % NI)

                if not skip_prev:
                    rpp, ipp = (to - 1) % NR, (to - 1) % NI
                    wait_rows(semg[rpp])
                    pltpu.async_copy(rows[rpp], acc_sh.at[ibuf[ipp].at[1]],
                                     sems[rpp], add=True)

            def final_scatter(to):
                rp, ip = to % NR, to % NI
                wait_rows(semg[rp])
                pltpu.async_copy(rows[rp], acc_sh.at[ibuf[ip].at[1]],
                                 sems[rp], add=True)

            issue_idx(0, 0)
            issue_idx(1, 1)
            do_block(0, 0, True, True)
            do_block(1, 1, True, False)
            do_block(2, 2, True, False)

            def main_body(it, carry):
                base = 3 + (NR * NI) * it
                for k in range(NR * NI):
                    do_block(base + k, 3 + k, False, False)
                return carry

            lax.fori_loop(0, main_iters, main_body, 0)

            for k in range(3):
                do_block(tail0 + k, tail0 + k, False, False)

            @pl.when(s < brem)
            def _():
                do_block(tail0 + 3, tail0 + 3, False, False)
                final_scatter(tail0 + 3)

            @pl.when(s >= brem)
            def _():
                final_scatter(tail0 + 2)

            for r in range(NR):
                wait_rows(sems[r])

        @pl.when(c == 0)
        def _():
            run_graph(hs_hbm, edm_hbm)

        @pl.when(c == 1)
        def _():
            run_graph(hr_hbm, edr_hbm)

        plsc.subcore_barrier()

        @pl.when(c == 0)
        def _():
            pltpu.sync_copy(acc_sh.at[pl.ds(s * rpt, rpt)],
                            aggs_hbm.at[pl.ds(s * rpt, rpt)])

            @pl.when(s == NS - 1)
            def _():
                pltpu.sync_copy(acc_sh.at[pl.ds(NS * rpt, tail)],
                                aggs_hbm.at[pl.ds(NS * rpt, tail)])

        @pl.when(c == 1)
        def _():
            pltpu.sync_copy(acc_sh.at[pl.ds(s * rpt, rpt)],
                            aggr_hbm.at[pl.ds(s * rpt, rpt)])

            @pl.when(s == NS - 1)
            def _():
                pltpu.sync_copy(acc_sh.at[pl.ds(NS * rpt, tail)],
                                aggr_hbm.at[pl.ds(NS * rpt, tail)])

    return scatter_kernel(hs, hr, edm, edr, zrows)


def _tc_pre(histp, x, sw1, rw1):
    n, d = x.shape
    g = n // ROW_BLK
    nt = NC * NS

    def body(hp_ref, x_ref, sw_ref, rw_ref, h1s_ref, h1r_ref, rsd_ref):
        deg = jnp.sum(hp_ref[...], axis=0)
        rsd = lax.rsqrt(jnp.maximum(deg, 1.0))
        rsd_ref[...] = rsd
        xs = x_ref[...]
        h1s_ref[...] = jnp.dot(xs * rsd[:, 0:1], sw_ref[...],
                               preferred_element_type=jnp.float32)
        h1r_ref[...] = jnp.dot(xs * rsd[:, 2:3], rw_ref[...],
                               preferred_element_type=jnp.float32)

    return pl.pallas_call(
        body,
        grid=(g,),
        in_specs=[
            pl.BlockSpec((nt, ROW_BLK, 4), lambda i: (0, i, 0)),
            pl.BlockSpec((ROW_BLK, d), lambda i: (i, 0)),
            pl.BlockSpec((d, d), lambda i: (0, 0)),
            pl.BlockSpec((d, d), lambda i: (0, 0)),
        ],
        out_specs=[
            pl.BlockSpec((ROW_BLK, d), lambda i: (i, 0)),
            pl.BlockSpec((ROW_BLK, d), lambda i: (i, 0)),
            pl.BlockSpec((ROW_BLK, 4), lambda i: (i, 0)),
        ],
        out_shape=[
            jax.ShapeDtypeStruct((n, d), jnp.float32),
            jax.ShapeDtypeStruct((n, d), jnp.float32),
            jax.ShapeDtypeStruct((n, 4), jnp.float32),
        ],
    )(histp, x, sw1, rw1)


def _elu(v):
    return jnp.where(v > 0, v, jnp.exp(jnp.minimum(v, 0.0)) - 1.0)


def _tc_stage_a(aggs, aggr, rsd, bs, br):
    n, d = aggs.shape
    g = n // ROW_BLK

    def body(as_ref, ar_ref, rsd_ref, bs_ref, br_ref, ys_ref, yr_ref, ps_ref):
        rsd = rsd_ref[...]
        ys = _elu(as_ref[...] * rsd[:, 1:2] + bs_ref[...])
        yr = _elu(ar_ref[...] * rsd[:, 3:4] + br_ref[...])
        ys_ref[...] = ys
        yr_ref[...] = yr
        ps_ref[...] = jnp.concatenate(
            [
                jnp.sum(ys, axis=0).reshape(1, 1, d),
                jnp.sum(ys * ys, axis=0).reshape(1, 1, d),
                jnp.sum(yr, axis=0).reshape(1, 1, d),
                jnp.sum(yr * yr, axis=0).reshape(1, 1, d),
            ],
            axis=1,
        )

    return pl.pallas_call(
        body,
        grid=(g,),
        in_specs=[
            pl.BlockSpec((ROW_BLK, d), lambda i: (i, 0)),
            pl.BlockSpec((ROW_BLK, d), lambda i: (i, 0)),
            pl.BlockSpec((ROW_BLK, 4), lambda i: (i, 0)),
            pl.BlockSpec((1, d), lambda i: (0, 0)),
            pl.BlockSpec((1, d), lambda i: (0, 0)),
        ],
        out_specs=[
            pl.BlockSpec((ROW_BLK, d), lambda i: (i, 0)),
            pl.BlockSpec((ROW_BLK, d), lambda i: (i, 0)),
            pl.BlockSpec((1, 4, d), lambda i: (i, 0, 0)),
        ],
        out_shape=[
            jax.ShapeDtypeStruct((n, d), jnp.float32),
            jax.ShapeDtypeStruct((n, d), jnp.float32),
            jax.ShapeDtypeStruct((g, 4, d), jnp.float32),
        ],
    )(aggs, aggr, rsd, bs, br)


def _bn_from_sums(y, tot_s1, tot_s2, n, gamma, beta):
    mu = tot_s1 / n
    var = tot_s2 / n - mu * mu
    return (y - mu) * lax.rsqrt(var + EPS) * gamma + beta


def _tc_mid_b(ys, yr, ps, rsd, gs, bes, gr, ber, sw2, rw2):
    n, d = ys.shape
    g = n // ROW_BLK

    def body(ys_ref, yr_ref, ps_ref, rsd_ref, gs_ref, bes_ref, gr_ref, ber_ref,
             sw_ref, rw_ref, h2s_ref, h2r_ref):
        tot = jnp.sum(ps_ref[...], axis=0)
        rsd = rsd_ref[...]
        zs = _bn_from_sums(ys_ref[...], tot[0:1], tot[1:2], n, gs_ref[...], bes_ref[...])
        zr = _bn_from_sums(yr_ref[...], tot[2:3], tot[3:4], n, gr_ref[...], ber_ref[...])
        h2s_ref[...] = jnp.dot(zs * rsd[:, 0:1], sw_ref[...],
                               preferred_element_type=jnp.float32)
        h2r_ref[...] = jnp.dot(zr * rsd[:, 2:3], rw_ref[...],
                               preferred_element_type=jnp.float32)

    return pl.pallas_call(
        body,
        grid=(g,),
        in_specs=[
            pl.BlockSpec((ROW_BLK, d), lambda i: (i, 0)),
            pl.BlockSpec((ROW_BLK, d), lambda i: (i, 0)),
            pl.BlockSpec((g, 4, d), lambda i: (0, 0, 0)),
            pl.BlockSpec((ROW_BLK, 4), lambda i: (i, 0)),
            pl.BlockSpec((1, d), lambda i: (0, 0)),
            pl.BlockSpec((1, d), lambda i: (0, 0)),
            pl.BlockSpec((1, d), lambda i: (0, 0)),
            pl.BlockSpec((1, d), lambda i: (0, 0)),
            pl.BlockSpec((d, d), lambda i: (0, 0)),
            pl.BlockSpec((d, d), lambda i: (0, 0)),
        ],
        out_specs=[
            pl.BlockSpec((ROW_BLK, d), lambda i: (i, 0)),
            pl.BlockSpec((ROW_BLK, d), lambda i: (i, 0)),
        ],
        out_shape=[
            jax.ShapeDtypeStruct((n, d), jnp.float32),
            jax.ShapeDtypeStruct((n, d), jnp.float32),
        ],
    )(ys, yr, ps, rsd, gs, bes, gr, ber, sw2, rw2)


def _tc_final(ys, yr, ps, gs, bes, gr, ber, g11, g12, g13, g14, gb1, pa, gw2,
              gb2, fw, fb):
    n, d = ys.shape
    c_out = fw.shape[1]
    g = n // ROW_BLK

    def body(ys_ref, yr_ref, ps_ref, gs_ref, bes_ref, gr_ref, ber_ref,
             g11_ref, g12_ref, g13_ref, g14_ref, gb1_ref, pa_ref, gw2_ref,
             gb2_ref, fw_ref, fb_ref, out_ref):
        tot = jnp.sum(ps_ref[...], axis=0)
        h2 = _bn_from_sums(ys_ref[...], tot[0:1], tot[1:2], n, gs_ref[...], bes_ref[...])
        h3 = _bn_from_sums(yr_ref[...], tot[2:3], tot[3:4], n, gr_ref[...], ber_ref[...])
        diff = jnp.abs(h2 - h3)
        prod = h2 * h3
        z = (jnp.dot(h2, g11_ref[...], preferred_element_type=jnp.float32)
             + jnp.dot(h3, g12_ref[...], preferred_element_type=jnp.float32)
             + jnp.dot(diff, g13_ref[...], preferred_element_type=jnp.float32)
             + jnp.dot(prod, g14_ref[...], preferred_element_type=jnp.float32)
             + gb1_ref[...])
        z = jnp.where(z > 0, z, pa_ref[...] * z)
        zz = jnp.dot(z, gw2_ref[...], preferred_element_type=jnp.float32) + gb2_ref[...]
        gate = 1.0 / (1.0 + jnp.exp(-zz))
        comb = gate * h2 + (1.0 - gate) * h3
        out_ref[...] = jnp.dot(comb, fw_ref[...],
                               preferred_element_type=jnp.float32) + fb_ref[...]

    return pl.pallas_call(
        body,
        grid=(g,),
        in_specs=[
            pl.BlockSpec((ROW_BLK, d), lambda i: (i, 0)),
            pl.BlockSpec((ROW_BLK, d), lambda i: (i, 0)),
            pl.BlockSpec((g, 4, d), lambda i: (0, 0, 0)),
            pl.BlockSpec((1, d), lambda i: (0, 0)),
            pl.BlockSpec((1, d), lambda i: (0, 0)),
            pl.BlockSpec((1, d), lambda i: (0, 0)),
            pl.BlockSpec((1, d), lambda i: (0, 0)),
            pl.BlockSpec((d, d), lambda i: (0, 0)),
            pl.BlockSpec((d, d), lambda i: (0, 0)),
            pl.BlockSpec((d, d), lambda i: (0, 0)),
            pl.BlockSpec((d, d), lambda i: (0, 0)),
            pl.BlockSpec((1, d), lambda i: (0, 0)),
            pl.BlockSpec((1, 1), lambda i: (0, 0)),
            pl.BlockSpec((d, d), lambda i: (0, 0)),
            pl.BlockSpec((1, d), lambda i: (0, 0)),
            pl.BlockSpec((d, c_out), lambda i: (0, 0)),
            pl.BlockSpec((1, c_out), lambda i: (0, 0)),
        ],
        out_specs=pl.BlockSpec((ROW_BLK, c_out), lambda i: (i, 0)),
        out_shape=jax.ShapeDtypeStruct((n, c_out), jnp.float32),
    )(ys, yr, ps, gs, bes, gr, ber, g11, g12, g13, g14, gb1, pa, gw2, gb2, fw, fb)


def kernel(node_features, mention_edges, retweet_edges, sW1, sb1, sg1, sbe1,
           sW2, sb2, sg2, sbe2, rW1, rb1, rg1, rbe1, rW2, rb2, rg2, rbe2,
           gW1, gb1, pa, gW2, gb2, fW, fb):
    n, d = node_features.shape
    srcm, dstm = mention_edges[0], mention_edges[1]
    srcr, dstr = retweet_edges[0], retweet_edges[1]
    edm = jnp.stack([srcm.reshape(-1, 128), dstm.reshape(-1, 128)], axis=1)
    edr = jnp.stack([srcr.reshape(-1, 128), dstr.reshape(-1, 128)], axis=1)

    histp = _sc_hist(srcm, dstm, srcr, dstr, n).reshape(NC * NS, n, 4)
    h1s, h1r, rsd = _tc_pre(histp, node_features, sW1, rW1)

    zrows = jnp.zeros((128, d), jnp.float32)
    agg1s, agg1r = _sc_scatter(h1s, h1r, edm, edr, zrows)
    ys1, yr1, ps1 = _tc_stage_a(agg1s, agg1r, rsd,
                                sb1.reshape(1, -1), rb1.reshape(1, -1))
    h2s, h2r = _tc_mid_b(ys1, yr1, ps1, rsd,
                         sg1.reshape(1, -1), sbe1.reshape(1, -1),
                         rg1.reshape(1, -1), rbe1.reshape(1, -1), sW2, rW2)

    agg2s, agg2r = _sc_scatter(h2s, h2r, edm, edr, zrows)
    ys2, yr2, ps2 = _tc_stage_a(agg2s, agg2r, rsd,
                                sb2.reshape(1, -1), rb2.reshape(1, -1))

    out = _tc_final(ys2, yr2, ps2,
                    sg2.reshape(1, -1), sbe2.reshape(1, -1),
                    rg2.reshape(1, -1), rbe2.reshape(1, -1),
                    gW1[0:d], gW1[d:2 * d], gW1[2 * d:3 * d], gW1[3 * d:4 * d],
                    gb1.reshape(1, -1), pa.reshape(1, 1), gW2,
                    gb2.reshape(1, -1), fW, fb.reshape(1, -1))
    return out


# speedup vs baseline: 8.8807x; 1.0044x over previous
import dataclasses
import functools

import jax
import jax.numpy as jnp
from jax import lax
from jax.experimental import pallas as pl
from jax.experimental.pallas import tpu as pltpu
from jax.experimental.pallas import tpu_sc as plsc

NC = 2
NS = 16
LANES = 16
EPS = 1e-5
ROW_BLK = 1000


def _vector_mesh():
    return plsc.VectorSubcoreMesh(
        core_axis_name="c", subcore_axis_name="s", num_cores=NC, num_subcores=NS
    )


def _sc_compiler_params():
    cp = pltpu.CompilerParams()
    if "needs_layout_passes" in pltpu.CompilerParams.__dataclass_fields__:
        cp = dataclasses.replace(cp, needs_layout_passes=False)
    return cp


def _sc_hist(srcm, dstm, srcr, dstr, n):
    e = srcm.shape[0]
    ec = e // (NC * NS)

    @functools.partial(
        pl.kernel,
        out_type=jax.ShapeDtypeStruct((NC * NS, 4 * n), jnp.float32),
        mesh=_vector_mesh(),
        scratch_types=[
            pltpu.VMEM((4 * n,), jnp.float32),
            pltpu.VMEM((ec,), jnp.int32),
        ],
        compiler_params=_sc_compiler_params(),
    )
    def hist_kernel(srcm_hbm, dstm_hbm, srcr_hbm, dstr_hbm, out_hbm, hist_v, idx_v):
        c = lax.axis_index("c")
        s = lax.axis_index("s")
        wid = c * NS + s

        zeros16 = jnp.zeros((LANES,), jnp.float32)

        @pl.loop(0, 4 * n, step=LANES)
        def _(i):
            hist_v[pl.ds(i, LANES)] = zeros16

        ones16 = jnp.ones((LANES,), jnp.float32)
        base = wid * ec
        for k, ref in enumerate((srcm_hbm, dstm_hbm, srcr_hbm, dstr_hbm)):
            pltpu.sync_copy(ref.at[pl.ds(base, ec)], idx_v)
            kvec = jnp.full((LANES,), k, jnp.int32)

            @pl.loop(0, ec, step=LANES)
            def _(i):
                idx = idx_v[pl.ds(i, LANES)]
                plsc.addupdate_scatter(hist_v, [idx * 4 + kvec], ones16)

        pltpu.sync_copy(hist_v, out_hbm.at[wid])

    return hist_kernel(srcm, dstm, srcr, dstr)


def _sc_scatter(hs, hr, srcm2, dstm2, srcr2, dstr2, zrows):
    n, h = hs.shape
    blk = 128
    nblk = srcm2.shape[0]
    bper = nblk // NS
    brem = nblk % NS
    NR = 3
    NI = 5
    main_iters = (bper - 3 - 3) // (NR * NI)
    tail0 = 3 + main_iters * NR * NI
    rpt = (n // NS) // 8 * 8
    tail = n - NS * rpt

    out_sds = jax.ShapeDtypeStruct((n, h), jnp.float32)

    @functools.partial(
        pl.kernel,
        out_type=(out_sds, out_sds),
        mesh=_vector_mesh(),
        scratch_types=(
            [pltpu.VMEM_SHARED((n, h), jnp.float32)]
            + [pltpu.VMEM((blk, h), jnp.float32)] * NR
            + [pltpu.VMEM((blk,), jnp.int32)] * (2 * NI)
            + [pltpu.SemaphoreType.DMA] * (2 * NR + NI)
        ),
    )
    def scatter_kernel(hs_hbm, hr_hbm, srcm_hbm, dstm_hbm, srcr_hbm, dstr_hbm,
                       zrows_hbm, aggs_hbm, aggr_hbm,
                       acc_sh, *bufs):
        c = lax.axis_index("c")
        s = lax.axis_index("s")
        rows = bufs[:NR]
        isrc = bufs[NR:NR + NI]
        idst = bufs[NR + NI:NR + 2 * NI]
        semg = bufs[NR + 2 * NI:2 * NR + 2 * NI]
        sems = bufs[2 * NR + 2 * NI:3 * NR + 2 * NI]
        semi = bufs[3 * NR + 2 * NI:3 * NR + 3 * NI]
        rows_a = rows[0]

        pltpu.sync_copy(zrows_hbm, rows_a)

        @pl.loop(0, 4)
        def _(i):
            pltpu.sync_copy(rows_a, acc_sh.at[pl.ds(s * rpt + i * blk, blk)])

        pltpu.sync_copy(rows_a.at[pl.ds(0, rpt - 4 * blk)],
                        acc_sh.at[pl.ds(s * rpt + 4 * blk, rpt - 4 * blk)])

        @pl.when(s == NS - 1)
        def _():
            pltpu.sync_copy(rows_a.at[pl.ds(0, tail)],
                            acc_sh.at[pl.ds(NS * rpt, tail)])

        plsc.subcore_barrier()

        count = bper + jnp.where(s < brem, 1, 0)

        def run_graph(h_hbm, src_hbm, dst_hbm):
            def issue_idx(t, p):
                gb = s + NS * t
                pltpu.async_copy(src_hbm.at[gb], isrc[p], semi[p])
                pltpu.async_copy(dst_hbm.at[gb], idst[p], semi[p])

            def wait_idx(p):
                pltpu.make_async_copy(src_hbm.at[0], isrc[p], semi[p]).wait()
                pltpu.make_async_copy(dst_hbm.at[0], idst[p], semi[p]).wait()

            def wait_rows(sem):
                pltpu.make_async_copy(hs_hbm.at[pl.ds(0, blk)], rows_a,
                                      sem).wait()

            def do_block(t, to, skip_wait, skip_prev):
                rp, ip = to % NR, to % NI
                if not skip_wait:
                    wait_rows(sems[rp])
                wait_idx(ip)
                pltpu.async_copy(h_hbm.at[isrc[ip]], rows[rp], semg[rp])

                @pl.when(t + 2 < count)
                def _():
                    issue_idx(t + 2, (to + 2) % NI)

                if not skip_prev:
                    rpp, ipp = (to - 1) % NR, (to - 1) % NI
                    wait_rows(semg[rpp])
                    pltpu.async_copy(rows[rpp], acc_sh.at[idst[ipp]],
                                     sems[rpp], add=True)

            def final_scatter(to):
                rp, ip = to % NR, to % NI
                wait_rows(semg[rp])
                pltpu.async_copy(rows[rp], acc_sh.at[idst[ip]],
                                 sems[rp], add=True)

            issue_idx(0, 0)
            issue_idx(1, 1)
            do_block(0, 0, True, True)
            do_block(1, 1, True, False)
            do_block(2, 2, True, False)

            def main_body(it, carry):
                base = 3 + (NR * NI) * it
                for k in range(NR * NI):
                    do_block(base + k, 3 + k, False, False)
                return carry

            lax.fori_loop(0, main_iters, main_body, 0)

            for k in range(3):
                do_block(tail0 + k, tail0 + k, False, False)

            @pl.when(s < brem)
            def _():
                do_block(tail0 + 3, tail0 + 3, False, False)
                final_scatter(tail0 + 3)

            @pl.when(s >= brem)
            def _():
                final_scatter(tail0 + 2)

            for r in range(NR):
                wait_rows(sems[r])

        @pl.when(c == 0)
        def _():
            run_graph(hs_hbm, srcm_hbm, dstm_hbm)

        @pl.when(c == 1)
        def _():
            run_graph(hr_hbm, srcr_hbm, dstr_hbm)

        plsc.subcore_barrier()

        @pl.when(c == 0)
        def _():
            pltpu.sync_copy(acc_sh.at[pl.ds(s * rpt, rpt)],
                            aggs_hbm.at[pl.ds(s * rpt, rpt)])

            @pl.when(s == NS - 1)
            def _():
                pltpu.sync_copy(acc_sh.at[pl.ds(NS * rpt, tail)],
                                aggs_hbm.at[pl.ds(NS * rpt, tail)])

        @pl.when(c == 1)
        def _():
            pltpu.sync_copy(acc_sh.at[pl.ds(s * rpt, rpt)],
                            aggr_hbm.at[pl.ds(s * rpt, rpt)])

            @pl.when(s == NS - 1)
            def _():
                pltpu.sync_copy(acc_sh.at[pl.ds(NS * rpt, tail)],
                                aggr_hbm.at[pl.ds(NS * rpt, tail)])

    return scatter_kernel(hs, hr, srcm2, dstm2, srcr2, dstr2, zrows)


def _tc_mm1(x, sw1, rw1):
    n, d = x.shape
    g = n // ROW_BLK

    def body(x_ref, sw_ref, rw_ref, h1s_ref, h1r_ref):
        xs = x_ref[...]
        h1s_ref[...] = jnp.dot(xs, sw_ref[...],
                               preferred_element_type=jnp.float32)
        h1r_ref[...] = jnp.dot(xs, rw_ref[...],
                               preferred_element_type=jnp.float32)

    return pl.pallas_call(
        body,
        grid=(g,),
        in_specs=[
            pl.BlockSpec((ROW_BLK, d), lambda i: (i, 0)),
            pl.BlockSpec((d, d), lambda i: (0, 0)),
            pl.BlockSpec((d, d), lambda i: (0, 0)),
        ],
        out_specs=[
            pl.BlockSpec((ROW_BLK, d), lambda i: (i, 0)),
            pl.BlockSpec((ROW_BLK, d), lambda i: (i, 0)),
        ],
        out_shape=[
            jax.ShapeDtypeStruct((n, d), jnp.float32),
            jax.ShapeDtypeStruct((n, d), jnp.float32),
        ],
    )(x, sw1, rw1)


def _tc_scale(histp, h1s, h1r):
    n, d = h1s.shape
    g = n // ROW_BLK
    nt = NC * NS

    def body(hp_ref, h1s_ref, h1r_ref, o1s_ref, o1r_ref, rsd_ref):
        deg = jnp.sum(hp_ref[...], axis=0)
        rsd = lax.rsqrt(jnp.maximum(deg, 1.0))
        rsd_ref[...] = rsd
        o1s_ref[...] = h1s_ref[...] * rsd[:, 0:1]
        o1r_ref[...] = h1r_ref[...] * rsd[:, 2:3]

    return pl.pallas_call(
        body,
        grid=(g,),
        in_specs=[
            pl.BlockSpec((nt, ROW_BLK, 4), lambda i: (0, i, 0)),
            pl.BlockSpec((ROW_BLK, d), lambda i: (i, 0)),
            pl.BlockSpec((ROW_BLK, d), lambda i: (i, 0)),
        ],
        out_specs=[
            pl.BlockSpec((ROW_BLK, d), lambda i: (i, 0)),
            pl.BlockSpec((ROW_BLK, d), lambda i: (i, 0)),
            pl.BlockSpec((ROW_BLK, 4), lambda i: (i, 0)),
        ],
        out_shape=[
            jax.ShapeDtypeStruct((n, d), jnp.float32),
            jax.ShapeDtypeStruct((n, d), jnp.float32),
            jax.ShapeDtypeStruct((n, 4), jnp.float32),
        ],
    )(histp, h1s, h1r)


def _elu(v):
    return jnp.where(v > 0, v, jnp.exp(jnp.minimum(v, 0.0)) - 1.0)


def _tc_stage_a(aggs, aggr, rsd, bs, br):
    n, d = aggs.shape
    g = n // ROW_BLK

    def body(as_ref, ar_ref, rsd_ref, bs_ref, br_ref, ys_ref, yr_ref, ps_ref):
        rsd = rsd_ref[...]
        ys = _elu(as_ref[...] * rsd[:, 1:2] + bs_ref[...][None, :])
        yr = _elu(ar_ref[...] * rsd[:, 3:4] + br_ref[...][None, :])
        ys_ref[...] = ys
        yr_ref[...] = yr
        ps_ref[...] = jnp.concatenate(
            [
                jnp.sum(ys, axis=0).reshape(1, 1, d),
                jnp.sum(ys * ys, axis=0).reshape(1, 1, d),
                jnp.sum(yr, axis=0).reshape(1, 1, d),
                jnp.sum(yr * yr, axis=0).reshape(1, 1, d),
            ],
            axis=1,
        )

    return pl.pallas_call(
        body,
        grid=(g,),
        in_specs=[
            pl.BlockSpec((ROW_BLK, d), lambda i: (i, 0)),
            pl.BlockSpec((ROW_BLK, d), lambda i: (i, 0)),
            pl.BlockSpec((ROW_BLK, 4), lambda i: (i, 0)),
            pl.BlockSpec((d,), lambda i: (0,)),
            pl.BlockSpec((d,), lambda i: (0,)),
        ],
        out_specs=[
            pl.BlockSpec((ROW_BLK, d), lambda i: (i, 0)),
            pl.BlockSpec((ROW_BLK, d), lambda i: (i, 0)),
            pl.BlockSpec((1, 4, d), lambda i: (i, 0, 0)),
        ],
        out_shape=[
            jax.ShapeDtypeStruct((n, d), jnp.float32),
            jax.ShapeDtypeStruct((n, d), jnp.float32),
            jax.ShapeDtypeStruct((g, 4, d), jnp.float32),
        ],
    )(aggs, aggr, rsd, bs, br)


def _bn_from_sums(y, tot_s1, tot_s2, n, gamma, beta):
    mu = tot_s1 / n
    var = tot_s2 / n - mu * mu
    return (y - mu) * lax.rsqrt(var + EPS) * gamma + beta


def _tc_mid_b(ys, yr, ps, rsd, gs, bes, gr, ber, sw2, rw2):
    n, d = ys.shape
    g = n // ROW_BLK

    def body(ys_ref, yr_ref, ps_ref, rsd_ref, gs_ref, bes_ref, gr_ref, ber_ref,
             sw_ref, rw_ref, h2s_ref, h2r_ref):
        tot = jnp.sum(ps_ref[...], axis=0)
        rsd = rsd_ref[...]
        zs = _bn_from_sums(ys_ref[...], tot[0:1], tot[1:2], n,
                           gs_ref[...][None, :], bes_ref[...][None, :])
        zr = _bn_from_sums(yr_ref[...], tot[2:3], tot[3:4], n,
                           gr_ref[...][None, :], ber_ref[...][None, :])
        h2s_ref[...] = jnp.dot(zs * rsd[:, 0:1], sw_ref[...],
                               preferred_element_type=jnp.float32)
        h2r_ref[...] = jnp.dot(zr * rsd[:, 2:3], rw_ref[...],
                               preferred_element_type=jnp.float32)

    return pl.pallas_call(
        body,
        grid=(g,),
        in_specs=[
            pl.BlockSpec((ROW_BLK, d), lambda i: (i, 0)),
            pl.BlockSpec((ROW_BLK, d), lambda i: (i, 0)),
            pl.BlockSpec((g, 4, d), lambda i: (0, 0, 0)),
            pl.BlockSpec((ROW_BLK, 4), lambda i: (i, 0)),
            pl.BlockSpec((d,), lambda i: (0,)),
            pl.BlockSpec((d,), lambda i: (0,)),
            pl.BlockSpec((d,), lambda i: (0,)),
            pl.BlockSpec((d,), lambda i: (0,)),
            pl.BlockSpec((d, d), lambda i: (0, 0)),
            pl.BlockSpec((d, d), lambda i: (0, 0)),
        ],
        out_specs=[
            pl.BlockSpec((ROW_BLK, d), lambda i: (i, 0)),
            pl.BlockSpec((ROW_BLK, d), lambda i: (i, 0)),
        ],
        out_shape=[
            jax.ShapeDtypeStruct((n, d), jnp.float32),
            jax.ShapeDtypeStruct((n, d), jnp.float32),
        ],
    )(ys, yr, ps, rsd, gs, bes, gr, ber, sw2, rw2)


def _tc_final(ys, yr, ps, gs, bes, gr, ber, gw1, gb1, pa, gw2, gb2, fw, fb):
    n, d = ys.shape
    c_out = fw.shape[1]
    g = n // ROW_BLK

    def body(ys_ref, yr_ref, ps_ref, gs_ref, bes_ref, gr_ref, ber_ref,
             gw1_ref, gb1_ref, pa_ref, gw2_ref, gb2_ref, fw_ref, fb_ref,
             out_ref):
        tot = jnp.sum(ps_ref[...], axis=0)
        h2 = _bn_from_sums(ys_ref[...], tot[0:1], tot[1:2], n,
                           gs_ref[...][None, :], bes_ref[...][None, :])
        h3 = _bn_from_sums(yr_ref[...], tot[2:3], tot[3:4], n,
                           gr_ref[...][None, :], ber_ref[...][None, :])
        diff = jnp.abs(h2 - h3)
        prod = h2 * h3
        z = (jnp.dot(h2, gw1_ref[0:d, :], preferred_element_type=jnp.float32)
             + jnp.dot(h3, gw1_ref[d:2 * d, :],
                       preferred_element_type=jnp.float32)
             + jnp.dot(diff, gw1_ref[2 * d:3 * d, :],
                       preferred_element_type=jnp.float32)
             + jnp.dot(prod, gw1_ref[3 * d:4 * d, :],
                       preferred_element_type=jnp.float32)
             + gb1_ref[...][None, :])
        z = jnp.where(z > 0, z, pa_ref[...] * z)
        zz = (jnp.dot(z, gw2_ref[...], preferred_element_type=jnp.float32)
              + gb2_ref[...][None, :])
        gate = 1.0 / (1.0 + jnp.exp(-zz))
        comb = gate * h2 + (1.0 - gate) * h3
        out_ref[...] = (jnp.dot(comb, fw_ref[...],
                                preferred_element_type=jnp.float32)
                        + fb_ref[...][None, :])

    return pl.pallas_call(
        body,
        grid=(g,),
        in_specs=[
            pl.BlockSpec((ROW_BLK, d), lambda i: (i, 0)),
            pl.BlockSpec((ROW_BLK, d), lambda i: (i, 0)),
            pl.BlockSpec((g, 4, d), lambda i: (0, 0, 0)),
            pl.BlockSpec((d,), lambda i: (0,)),
            pl.BlockSpec((d,), lambda i: (0,)),
            pl.BlockSpec((d,), lambda i: (0,)),
            pl.BlockSpec((d,), lambda i: (0,)),
            pl.BlockSpec((4 * d, d), lambda i: (0, 0)),
            pl.BlockSpec((d,), lambda i: (0,)),
            pl.BlockSpec((1, 1), lambda i: (0, 0)),
            pl.BlockSpec((d, d), lambda i: (0, 0)),
            pl.BlockSpec((d,), lambda i: (0,)),
            pl.BlockSpec((d, c_out), lambda i: (0, 0)),
            pl.BlockSpec((c_out,), lambda i: (0,)),
        ],
        out_specs=pl.BlockSpec((ROW_BLK, c_out), lambda i: (i, 0)),
        out_shape=jax.ShapeDtypeStruct((n, c_out), jnp.float32),
    )(ys, yr, ps, gs, bes, gr, ber, gw1, gb1, pa, gw2, gb2, fw, fb)


def kernel(node_features, mention_edges, retweet_edges, sW1, sb1, sg1, sbe1,
           sW2, sb2, sg2, sbe2, rW1, rb1, rg1, rbe1, rW2, rb2, rg2, rbe2,
           gW1, gb1, pa, gW2, gb2, fW, fb):
    n, d = node_features.shape
    srcm, dstm = mention_edges[0], mention_edges[1]
    srcr, dstr = retweet_edges[0], retweet_edges[1]
    srcm2, dstm2 = srcm.reshape(-1, 128), dstm.reshape(-1, 128)
    srcr2, dstr2 = srcr.reshape(-1, 128), dstr.reshape(-1, 128)

    histp = _sc_hist(srcm, dstm, srcr, dstr, n).reshape(NC * NS, n, 4)
    h1s0, h1r0 = _tc_mm1(node_features, sW1, rW1)
    h1s, h1r, rsd = _tc_scale(histp, h1s0, h1r0)

    zrows = jnp.zeros((128, d), jnp.float32)
    agg1s, agg1r = _sc_scatter(h1s, h1r, srcm2, dstm2, srcr2, dstr2, zrows)
    ys1, yr1, ps1 = _tc_stage_a(agg1s, agg1r, rsd, sb1, rb1)
    h2s, h2r = _tc_mid_b(ys1, yr1, ps1, rsd, sg1, sbe1, rg1, rbe1, sW2, rW2)

    agg2s, agg2r = _sc_scatter(h2s, h2r, srcm2, dstm2, srcr2, dstr2, zrows)
    ys2, yr2, ps2 = _tc_stage_a(agg2s, agg2r, rsd, sb2, rb2)

    return _tc_final(ys2, yr2, ps2, sg2, sbe2, rg2, rbe2,
                     gW1, gb1, pa.reshape(1, 1), gW2, gb2, fW, fb)


# speedup vs baseline: 10.9930x; 1.2378x over previous
import dataclasses
import functools

import jax
import jax.numpy as jnp
from jax import lax
from jax.experimental import pallas as pl
from jax.experimental.pallas import tpu as pltpu
from jax.experimental.pallas import tpu_sc as plsc

NC = 2
NS = 16
LANES = 16
EPS = 1e-5
ROW_BLK = 1000


def _vector_mesh():
    return plsc.VectorSubcoreMesh(
        core_axis_name="c", subcore_axis_name="s", num_cores=NC, num_subcores=NS
    )


def _sc_compiler_params():
    cp = pltpu.CompilerParams()
    if "needs_layout_passes" in pltpu.CompilerParams.__dataclass_fields__:
        cp = dataclasses.replace(cp, needs_layout_passes=False)
    return cp


def _sc_hist(srcm, dstm, srcr, dstr, iota3, zrows, n):
    e = srcm.shape[0]
    ec = e // (NC * NS)
    hr = (4 * n + 127) // 128
    hrp = (hr + 127) // 128 * 128
    rpt = hrp // NS

    @functools.partial(
        pl.kernel,
        out_type=jax.ShapeDtypeStruct((NC, hrp, 128), jnp.float32),
        mesh=_vector_mesh(),
        scratch_types=[
            pltpu.VMEM_SHARED((hrp, 128), jnp.float32),
            pltpu.VMEM((hrp, 128), jnp.float32),
            pltpu.VMEM((ec,), jnp.int32),
            pltpu.VMEM((hrp // 128, 128), jnp.int32),
        ],
        compiler_params=_sc_compiler_params(),
    )
    def hist_kernel(srcm_hbm, dstm_hbm, srcr_hbm, dstr_hbm, iota_hbm,
                    zrows_hbm, out_hbm, acc_sh, hist_v, idx_v, iota_v):
        c = lax.axis_index("c")
        s = lax.axis_index("s")
        wid = c * NS + s

        for j in range(hrp // 128):
            pltpu.sync_copy(zrows_hbm, hist_v.at[pl.ds(j * 128, 128)])
        pltpu.sync_copy(iota_hbm, iota_v)
        pltpu.sync_copy(hist_v.at[pl.ds(0, rpt)],
                        acc_sh.at[pl.ds(s * rpt, rpt)])

        ones16 = jnp.ones((LANES,), jnp.float32)
        base = wid * ec
        for k, ref in enumerate((srcm_hbm, dstm_hbm, srcr_hbm, dstr_hbm)):
            pltpu.sync_copy(ref.at[pl.ds(base, ec)], idx_v)
            kvec = jnp.full((LANES,), k, jnp.int32)

            @pl.loop(0, ec, step=LANES)
            def _(i):
                idx = idx_v[pl.ds(i, LANES)] * 4 + kvec
                plsc.addupdate_scatter(
                    hist_v,
                    [lax.shift_right_logical(idx, 7),
                     lax.bitwise_and(idx, 127)],
                    ones16)

        plsc.subcore_barrier()
        for j in range(hrp // 128):
            pltpu.sync_copy(hist_v.at[pl.ds(j * 128, 128)],
                            acc_sh.at[iota_v.at[j]], add=True)
        plsc.subcore_barrier()

        pltpu.sync_copy(acc_sh.at[pl.ds(s * rpt, rpt)],
                        out_hbm.at[c, pl.ds(s * rpt, rpt)])

    return hist_kernel(srcm, dstm, srcr, dstr, iota3, zrows)


def _sc_scatter(hs, hr, srcm2, dstm2, srcr2, dstr2, zrows):
    n, h = hs.shape
    blk = 128
    nblk = srcm2.shape[0]
    bper = nblk // NS
    brem = nblk % NS
    NR = 3
    NI = 5
    main_iters = (bper - 3 - 3) // (NR * NI)
    tail0 = 3 + main_iters * NR * NI
    rpt = (n // NS) // 8 * 8
    tail = n - NS * rpt

    out_sds = jax.ShapeDtypeStruct((n, h), jnp.float32)

    @functools.partial(
        pl.kernel,
        out_type=(out_sds, out_sds),
        mesh=_vector_mesh(),
        scratch_types=(
            [pltpu.VMEM_SHARED((n, h), jnp.float32)]
            + [pltpu.VMEM((blk, h), jnp.float32)] * NR
            + [pltpu.VMEM((blk,), jnp.int32)] * (2 * NI)
            + [pltpu.SemaphoreType.DMA] * (2 * NR + NI)
        ),
    )
    def scatter_kernel(hs_hbm, hr_hbm, srcm_hbm, dstm_hbm, srcr_hbm, dstr_hbm,
                       zrows_hbm, aggs_hbm, aggr_hbm,
                       acc_sh, *bufs):
        c = lax.axis_index("c")
        s = lax.axis_index("s")
        rows = bufs[:NR]
        isrc = bufs[NR:NR + NI]
        idst = bufs[NR + NI:NR + 2 * NI]
        semg = bufs[NR + 2 * NI:2 * NR + 2 * NI]
        sems = bufs[2 * NR + 2 * NI:3 * NR + 2 * NI]
        semi = bufs[3 * NR + 2 * NI:3 * NR + 3 * NI]
        rows_a = rows[0]

        pltpu.sync_copy(zrows_hbm, rows_a)

        @pl.loop(0, 4)
        def _(i):
            pltpu.sync_copy(rows_a, acc_sh.at[pl.ds(s * rpt + i * blk, blk)])

        pltpu.sync_copy(rows_a.at[pl.ds(0, rpt - 4 * blk)],
                        acc_sh.at[pl.ds(s * rpt + 4 * blk, rpt - 4 * blk)])

        @pl.when(s == NS - 1)
        def _():
            pltpu.sync_copy(rows_a.at[pl.ds(0, tail)],
                            acc_sh.at[pl.ds(NS * rpt, tail)])

        plsc.subcore_barrier()

        count = bper + jnp.where(s < brem, 1, 0)

        def run_graph(h_hbm, src_hbm, dst_hbm):
            def issue_idx(t, p):
                gb = s + NS * t
                pltpu.async_copy(src_hbm.at[gb], isrc[p], semi[p])
                pltpu.async_copy(dst_hbm.at[gb], idst[p], semi[p])

            def wait_idx(p):
                pltpu.make_async_copy(src_hbm.at[0], isrc[p], semi[p]).wait()
                pltpu.make_async_copy(dst_hbm.at[0], idst[p], semi[p]).wait()

            def wait_rows(sem):
                pltpu.make_async_copy(hs_hbm.at[pl.ds(0, blk)], rows_a,
                                      sem).wait()

            def do_block(t, to, skip_wait, skip_prev):
                rp, ip = to % NR, to % NI
                if not skip_wait:
                    wait_rows(sems[rp])
                wait_idx(ip)
                pltpu.async_copy(h_hbm.at[isrc[ip]], rows[rp], semg[rp])

                @pl.when(t + 2 < count)
                def _():
                    issue_idx(t + 2, (to + 2) % NI)

                if not skip_prev:
                    rpp, ipp = (to - 1) % NR, (to - 1) % NI
                    wait_rows(semg[rpp])
                    pltpu.async_copy(rows[rpp], acc_sh.at[idst[ipp]],
                                     sems[rpp], add=True)

            def final_scatter(to):
                rp, ip = to % NR, to % NI
                wait_rows(semg[rp])
                pltpu.async_copy(rows[rp], acc_sh.at[idst[ip]],
                                 sems[rp], add=True)

            issue_idx(0, 0)
            issue_idx(1, 1)
            do_block(0, 0, True, True)
            do_block(1, 1, True, False)
            do_block(2, 2, True, False)

            def main_body(it, carry):
                base = 3 + (NR * NI) * it
                for k in range(NR * NI):
                    do_block(base + k, 3 + k, False, False)
                return carry

            lax.fori_loop(0, main_iters, main_body, 0)

            for k in range(3):
                do_block(tail0 + k, tail0 + k, False, False)

            @pl.when(s < brem)
            def _():
                do_block(tail0 + 3, tail0 + 3, False, False)
                final_scatter(tail0 + 3)

            @pl.when(s >= brem)
            def _():
                final_scatter(tail0 + 2)

            for r in range(NR):
                wait_rows(sems[r])

        @pl.when(c == 0)
        def _():
            run_graph(hs_hbm, srcm_hbm, dstm_hbm)

        @pl.when(c == 1)
        def _():
            run_graph(hr_hbm, srcr_hbm, dstr_hbm)

        plsc.subcore_barrier()

        @pl.when(c == 0)
        def _():
            pltpu.sync_copy(acc_sh.at[pl.ds(s * rpt, rpt)],
                            aggs_hbm.at[pl.ds(s * rpt, rpt)])

            @pl.when(s == NS - 1)
            def _():
                pltpu.sync_copy(acc_sh.at[pl.ds(NS * rpt, tail)],
                                aggs_hbm.at[pl.ds(NS * rpt, tail)])

        @pl.when(c == 1)
        def _():
            pltpu.sync_copy(acc_sh.at[pl.ds(s * rpt, rpt)],
                            aggr_hbm.at[pl.ds(s * rpt, rpt)])

            @pl.when(s == NS - 1)
            def _():
                pltpu.sync_copy(acc_sh.at[pl.ds(NS * rpt, tail)],
                                aggr_hbm.at[pl.ds(NS * rpt, tail)])

    return scatter_kernel(hs, hr, srcm2, dstm2, srcr2, dstr2, zrows)


def _tc_mm1(x, sw1, rw1):
    n, d = x.shape
    g = n // ROW_BLK

    def body(x_ref, sw_ref, rw_ref, h1s_ref, h1r_ref):
        xs = x_ref[...]
        h1s_ref[...] = jnp.dot(xs, sw_ref[...],
                               preferred_element_type=jnp.float32)
        h1r_ref[...] = jnp.dot(xs, rw_ref[...],
                               preferred_element_type=jnp.float32)

    return pl.pallas_call(
        body,
        grid=(g,),
        in_specs=[
            pl.BlockSpec((ROW_BLK, d), lambda i: (i, 0)),
            pl.BlockSpec((d, d), lambda i: (0, 0)),
            pl.BlockSpec((d, d), lambda i: (0, 0)),
        ],
        out_specs=[
            pl.BlockSpec((ROW_BLK, d), lambda i: (i, 0)),
            pl.BlockSpec((ROW_BLK, d), lambda i: (i, 0)),
        ],
        out_shape=[
            jax.ShapeDtypeStruct((n, d), jnp.float32),
            jax.ShapeDtypeStruct((n, d), jnp.float32),
        ],
    )(x, sw1, rw1)


def _tc_scale(histp, h1s, h1r):
    n, d = h1s.shape
    g = n // ROW_BLK
    nt = histp.shape[0]

    def body(hp_ref, h1s_ref, h1r_ref, o1s_ref, o1r_ref, rsd_ref):
        deg = jnp.sum(hp_ref[...], axis=0)
        rsd = lax.rsqrt(jnp.maximum(deg, 1.0))
        rsd_ref[...] = rsd
        o1s_ref[...] = h1s_ref[...] * rsd[:, 0:1]
        o1r_ref[...] = h1r_ref[...] * rsd[:, 2:3]

    return pl.pallas_call(
        body,
        grid=(g,),
        in_specs=[
            pl.BlockSpec((nt, ROW_BLK, 4), lambda i: (0, i, 0)),
            pl.BlockSpec((ROW_BLK, d), lambda i: (i, 0)),
            pl.BlockSpec((ROW_BLK, d), lambda i: (i, 0)),
        ],
        out_specs=[
            pl.BlockSpec((ROW_BLK, d), lambda i: (i, 0)),
            pl.BlockSpec((ROW_BLK, d), lambda i: (i, 0)),
            pl.BlockSpec((ROW_BLK, 4), lambda i: (i, 0)),
        ],
        out_shape=[
            jax.ShapeDtypeStruct((n, d), jnp.float32),
            jax.ShapeDtypeStruct((n, d), jnp.float32),
            jax.ShapeDtypeStruct((n, 4), jnp.float32),
        ],
    )(histp, h1s, h1r)


def _elu(v):
    return jnp.where(v > 0, v, jnp.exp(jnp.minimum(v, 0.0)) - 1.0)


def _tc_stage_a(aggs, aggr, rsd, bs, br):
    n, d = aggs.shape
    g = n // ROW_BLK

    def body(as_ref, ar_ref, rsd_ref, bs_ref, br_ref, ys_ref, yr_ref, ps_ref):
        rsd = rsd_ref[...]
        ys = _elu(as_ref[...] * rsd[:, 1:2] + bs_ref[...][None, :])
        yr = _elu(ar_ref[...] * rsd[:, 3:4] + br_ref[...][None, :])
        ys_ref[...] = ys
        yr_ref[...] = yr
        ps_ref[...] = jnp.concatenate(
            [
                jnp.sum(ys, axis=0).reshape(1, 1, d),
                jnp.sum(ys * ys, axis=0).reshape(1, 1, d),
                jnp.sum(yr, axis=0).reshape(1, 1, d),
                jnp.sum(yr * yr, axis=0).reshape(1, 1, d),
            ],
            axis=1,
        )

    return pl.pallas_call(
        body,
        grid=(g,),
        in_specs=[
            pl.BlockSpec((ROW_BLK, d), lambda i: (i, 0)),
            pl.BlockSpec((ROW_BLK, d), lambda i: (i, 0)),
            pl.BlockSpec((ROW_BLK, 4), lambda i: (i, 0)),
            pl.BlockSpec((d,), lambda i: (0,)),
            pl.BlockSpec((d,), lambda i: (0,)),
        ],
        out_specs=[
            pl.BlockSpec((ROW_BLK, d), lambda i: (i, 0)),
            pl.BlockSpec((ROW_BLK, d), lambda i: (i, 0)),
            pl.BlockSpec((1, 4, d), lambda i: (i, 0, 0)),
        ],
        out_shape=[
            jax.ShapeDtypeStruct((n, d), jnp.float32),
            jax.ShapeDtypeStruct((n, d), jnp.float32),
            jax.ShapeDtypeStruct((g, 4, d), jnp.float32),
        ],
    )(aggs, aggr, rsd, bs, br)


def _bn_from_sums(y, tot_s1, tot_s2, n, gamma, beta):
    mu = tot_s1 / n
    var = tot_s2 / n - mu * mu
    return (y - mu) * lax.rsqrt(var + EPS) * gamma + beta


def _tc_mid_b(ys, yr, ps, rsd, gs, bes, gr, ber, sw2, rw2):
    n, d = ys.shape
    g = n // ROW_BLK

    def body(ys_ref, yr_ref, ps_ref, rsd_ref, gs_ref, bes_ref, gr_ref, ber_ref,
             sw_ref, rw_ref, h2s_ref, h2r_ref):
        tot = jnp.sum(ps_ref[...], axis=0)
        rsd = rsd_ref[...]
        zs = _bn_from_sums(ys_ref[...], tot[0:1], tot[1:2], n,
                           gs_ref[...][None, :], bes_ref[...][None, :])
        zr = _bn_from_sums(yr_ref[...], tot[2:3], tot[3:4], n,
                           gr_ref[...][None, :], ber_ref[...][None, :])
        h2s_ref[...] = jnp.dot(zs * rsd[:, 0:1], sw_ref[...],
                               preferred_element_type=jnp.float32)
        h2r_ref[...] = jnp.dot(zr * rsd[:, 2:3], rw_ref[...],
                               preferred_element_type=jnp.float32)

    return pl.pallas_call(
        body,
        grid=(g,),
        in_specs=[
            pl.BlockSpec((ROW_BLK, d), lambda i: (i, 0)),
            pl.BlockSpec((ROW_BLK, d), lambda i: (i, 0)),
            pl.BlockSpec((g, 4, d), lambda i: (0, 0, 0)),
            pl.BlockSpec((ROW_BLK, 4), lambda i: (i, 0)),
            pl.BlockSpec((d,), lambda i: (0,)),
            pl.BlockSpec((d,), lambda i: (0,)),
            pl.BlockSpec((d,), lambda i: (0,)),
            pl.BlockSpec((d,), lambda i: (0,)),
            pl.BlockSpec((d, d), lambda i: (0, 0)),
            pl.BlockSpec((d, d), lambda i: (0, 0)),
        ],
        out_specs=[
            pl.BlockSpec((ROW_BLK, d), lambda i: (i, 0)),
            pl.BlockSpec((ROW_BLK, d), lambda i: (i, 0)),
        ],
        out_shape=[
            jax.ShapeDtypeStruct((n, d), jnp.float32),
            jax.ShapeDtypeStruct((n, d), jnp.float32),
        ],
    )(ys, yr, ps, rsd, gs, bes, gr, ber, sw2, rw2)


def _tc_final(ys, yr, ps, gs, bes, gr, ber, gw1, gb1, pa, gw2, gb2, fw, fb):
    n, d = ys.shape
    c_out = fw.shape[1]
    g = n // ROW_BLK

    def body(ys_ref, yr_ref, ps_ref, gs_ref, bes_ref, gr_ref, ber_ref,
             gw1_ref, gb1_ref, pa_ref, gw2_ref, gb2_ref, fw_ref, fb_ref,
             out_ref):
        tot = jnp.sum(ps_ref[...], axis=0)
        h2 = _bn_from_sums(ys_ref[...], tot[0:1], tot[1:2], n,
                           gs_ref[...][None, :], bes_ref[...][None, :])
        h3 = _bn_from_sums(yr_ref[...], tot[2:3], tot[3:4], n,
                           gr_ref[...][None, :], ber_ref[...][None, :])
        diff = jnp.abs(h2 - h3)
        prod = h2 * h3
        z = (jnp.dot(h2, gw1_ref[0:d, :], preferred_element_type=jnp.float32)
             + jnp.dot(h3, gw1_ref[d:2 * d, :],
                       preferred_element_type=jnp.float32)
             + jnp.dot(diff, gw1_ref[2 * d:3 * d, :],
                       preferred_element_type=jnp.float32)
             + jnp.dot(prod, gw1_ref[3 * d:4 * d, :],
                       preferred_element_type=jnp.float32)
             + gb1_ref[...][None, :])
        z = jnp.where(z > 0, z, pa_ref[...] * z)
        zz = (jnp.dot(z, gw2_ref[...], preferred_element_type=jnp.float32)
              + gb2_ref[...][None, :])
        gate = 1.0 / (1.0 + jnp.exp(-zz))
        comb = gate * h2 + (1.0 - gate) * h3
        out_ref[...] = (jnp.dot(comb, fw_ref[...],
                                preferred_element_type=jnp.float32)
                        + fb_ref[...][None, :])

    return pl.pallas_call(
        body,
        grid=(g,),
        in_specs=[
            pl.BlockSpec((ROW_BLK, d), lambda i: (i, 0)),
            pl.BlockSpec((ROW_BLK, d), lambda i: (i, 0)),
            pl.BlockSpec((g, 4, d), lambda i: (0, 0, 0)),
            pl.BlockSpec((d,), lambda i: (0,)),
            pl.BlockSpec((d,), lambda i: (0,)),
            pl.BlockSpec((d,), lambda i: (0,)),
            pl.BlockSpec((d,), lambda i: (0,)),
            pl.BlockSpec((4 * d, d), lambda i: (0, 0)),
            pl.BlockSpec((d,), lambda i: (0,)),
            pl.BlockSpec((1, 1), lambda i: (0, 0)),
            pl.BlockSpec((d, d), lambda i: (0, 0)),
            pl.BlockSpec((d,), lambda i: (0,)),
            pl.BlockSpec((d, c_out), lambda i: (0, 0)),
            pl.BlockSpec((c_out,), lambda i: (0,)),
        ],
        out_specs=pl.BlockSpec((ROW_BLK, c_out), lambda i: (i, 0)),
        out_shape=jax.ShapeDtypeStruct((n, c_out), jnp.float32),
    )(ys, yr, ps, gs, bes, gr, ber, gw1, gb1, pa, gw2, gb2, fw, fb)


def kernel(node_features, mention_edges, retweet_edges, sW1, sb1, sg1, sbe1,
           sW2, sb2, sg2, sbe2, rW1, rb1, rg1, rbe1, rW2, rb2, rg2, rbe2,
           gW1, gb1, pa, gW2, gb2, fW, fb):
    n, d = node_features.shape
    srcm, dstm = mention_edges[0], mention_edges[1]
    srcr, dstr = retweet_edges[0], retweet_edges[1]
    srcm2, dstm2 = srcm.reshape(-1, 128), dstm.reshape(-1, 128)
    srcr2, dstr2 = srcr.reshape(-1, 128), dstr.reshape(-1, 128)

    zrows = jnp.zeros((128, d), jnp.float32)
    hrp = ((4 * n + 127) // 128 + 127) // 128 * 128
    iota3 = jnp.arange(hrp, dtype=jnp.int32).reshape(hrp // 128, 128)
    histp = _sc_hist(srcm, dstm, srcr, dstr, iota3, zrows, n)
    histp = histp.reshape(NC, hrp * 128)[:, :4 * n].reshape(NC, n, 4)
    h1s0, h1r0 = _tc_mm1(node_features, sW1, rW1)
    h1s, h1r, rsd = _tc_scale(histp, h1s0, h1r0)
    agg1s, agg1r = _sc_scatter(h1s, h1r, srcm2, dstm2, srcr2, dstr2, zrows)
    ys1, yr1, ps1 = _tc_stage_a(agg1s, agg1r, rsd, sb1, rb1)
    h2s, h2r = _tc_mid_b(ys1, yr1, ps1, rsd, sg1, sbe1, rg1, rbe1, sW2, rW2)

    agg2s, agg2r = _sc_scatter(h2s, h2r, srcm2, dstm2, srcr2, dstr2, zrows)
    ys2, yr2, ps2 = _tc_stage_a(agg2s, agg2r, rsd, sb2, rb2)

    return _tc_final(ys2, yr2, ps2, sg2, sbe2, rg2, rbe2,
                     gW1, gb1, pa.reshape(1, 1), gW2, gb2, fW, fb)


# speedup vs baseline: 11.0546x; 1.0056x over previous
import dataclasses
import functools

import jax
import jax.numpy as jnp
from jax import lax
from jax.experimental import pallas as pl
from jax.experimental.pallas import tpu as pltpu
from jax.experimental.pallas import tpu_sc as plsc

NC = 2
NS = 16
LANES = 16
EPS = 1e-5
ROW_BLK = 1000


def _vector_mesh():
    return plsc.VectorSubcoreMesh(
        core_axis_name="c", subcore_axis_name="s", num_cores=NC, num_subcores=NS
    )


def _sc_compiler_params():
    cp = pltpu.CompilerParams()
    if "needs_layout_passes" in pltpu.CompilerParams.__dataclass_fields__:
        cp = dataclasses.replace(cp, needs_layout_passes=False)
    return cp


def _sc_hist(srcm, dstm, srcr, dstr, iota3, zrows, n):
    e = srcm.shape[0]
    ec = e // (NC * NS)
    hr = (4 * n + 127) // 128
    hrp = (hr + 127) // 128 * 128
    rpt = hrp // NS

    @functools.partial(
        pl.kernel,
        out_type=jax.ShapeDtypeStruct((NC, hrp, 128), jnp.float32),
        mesh=_vector_mesh(),
        scratch_types=[
            pltpu.VMEM_SHARED((hrp, 128), jnp.float32),
            pltpu.VMEM((hrp, 128), jnp.float32),
            pltpu.VMEM((ec,), jnp.int32),
            pltpu.VMEM((hrp // 128, 128), jnp.int32),
        ],
        compiler_params=_sc_compiler_params(),
    )
    def hist_kernel(srcm_hbm, dstm_hbm, srcr_hbm, dstr_hbm, iota_hbm,
                    zrows_hbm, out_hbm, acc_sh, hist_v, idx_v, iota_v):
        c = lax.axis_index("c")
        s = lax.axis_index("s")
        wid = c * NS + s

        for j in range(hrp // 128):
            pltpu.sync_copy(zrows_hbm, hist_v.at[pl.ds(j * 128, 128)])
        pltpu.sync_copy(iota_hbm, iota_v)
        pltpu.sync_copy(hist_v.at[pl.ds(0, rpt)],
                        acc_sh.at[pl.ds(s * rpt, rpt)])

        ones16 = jnp.ones((LANES,), jnp.float32)
        base = wid * ec
        for k, ref in enumerate((srcm_hbm, dstm_hbm, srcr_hbm, dstr_hbm)):
            pltpu.sync_copy(ref.at[pl.ds(base, ec)], idx_v)
            kvec = jnp.full((LANES,), k, jnp.int32)

            @pl.loop(0, ec, step=5 * LANES)
            def _(i):
                for u in range(5):
                    idx = idx_v[pl.ds(i + u * LANES, LANES)] * 4 + kvec
                    plsc.addupdate_scatter(
                        hist_v,
                        [lax.shift_right_logical(idx, 7),
                         lax.bitwise_and(idx, 127)],
                        ones16)

        plsc.subcore_barrier()
        for j in range(hrp // 128):
            pltpu.sync_copy(hist_v.at[pl.ds(j * 128, 128)],
                            acc_sh.at[iota_v.at[j]], add=True)
        plsc.subcore_barrier()

        pltpu.sync_copy(acc_sh.at[pl.ds(s * rpt, rpt)],
                        out_hbm.at[c, pl.ds(s * rpt, rpt)])

    return hist_kernel(srcm, dstm, srcr, dstr, iota3, zrows)


def _sc_scatter(hs, hr, srcm2, dstm2, srcr2, dstr2, zrows):
    n, h = hs.shape
    blk = 128
    nblk = srcm2.shape[0]
    bper = nblk // NS
    brem = nblk % NS
    NR = 3
    NI = 5
    main_iters = (bper - 3 - 3) // (NR * NI)
    tail0 = 3 + main_iters * NR * NI
    rpt = (n // NS) // 8 * 8
    tail = n - NS * rpt

    out_sds = jax.ShapeDtypeStruct((n, h), jnp.float32)

    @functools.partial(
        pl.kernel,
        out_type=(out_sds, out_sds),
        mesh=_vector_mesh(),
        scratch_types=(
            [pltpu.VMEM_SHARED((n, h), jnp.float32)]
            + [pltpu.VMEM((blk, h), jnp.float32)] * NR
            + [pltpu.VMEM((blk,), jnp.int32)] * (2 * NI)
            + [pltpu.SemaphoreType.DMA] * (2 * NR + NI)
        ),
    )
    def scatter_kernel(hs_hbm, hr_hbm, srcm_hbm, dstm_hbm, srcr_hbm, dstr_hbm,
                       zrows_hbm, aggs_hbm, aggr_hbm,
                       acc_sh, *bufs):
        c = lax.axis_index("c")
        s = lax.axis_index("s")
        rows = bufs[:NR]
        isrc = bufs[NR:NR + NI]
        idst = bufs[NR + NI:NR + 2 * NI]
        semg = bufs[NR + 2 * NI:2 * NR + 2 * NI]
        sems = bufs[2 * NR + 2 * NI:3 * NR + 2 * NI]
        semi = bufs[3 * NR + 2 * NI:3 * NR + 3 * NI]
        rows_a = rows[0]

        pltpu.sync_copy(zrows_hbm, rows_a)

        descs = [
            pltpu.async_copy(rows_a, acc_sh.at[pl.ds(s * rpt + i * blk, blk)],
                             semg[0])
            for i in range(4)
        ]
        descs.append(
            pltpu.async_copy(rows_a.at[pl.ds(0, rpt - 4 * blk)],
                             acc_sh.at[pl.ds(s * rpt + 4 * blk,
                                             rpt - 4 * blk)], semg[0]))

        @pl.when(s == NS - 1)
        def _():
            pltpu.async_copy(rows_a.at[pl.ds(0, tail)],
                             acc_sh.at[pl.ds(NS * rpt, tail)],
                             semg[0]).wait()

        for d_ in descs:
            d_.wait()

        plsc.subcore_barrier()

        count = bper + jnp.where(s < brem, 1, 0)

        def run_graph(h_hbm, src_hbm, dst_hbm):
            def issue_idx(t, p):
                gb = s + NS * t
                pltpu.async_copy(src_hbm.at[gb], isrc[p], semi[p])
                pltpu.async_copy(dst_hbm.at[gb], idst[p], semi[p])

            def wait_idx(p):
                pltpu.make_async_copy(src_hbm.at[0], isrc[p], semi[p]).wait()
                pltpu.make_async_copy(dst_hbm.at[0], idst[p], semi[p]).wait()

            def wait_rows(sem):
                pltpu.make_async_copy(hs_hbm.at[pl.ds(0, blk)], rows_a,
                                      sem).wait()

            def do_block(t, to, skip_wait, skip_prev):
                rp, ip = to % NR, to % NI
                if not skip_wait:
                    wait_rows(sems[rp])
                wait_idx(ip)
                pltpu.async_copy(h_hbm.at[isrc[ip]], rows[rp], semg[rp])

                @pl.when(t + 2 < count)
                def _():
                    issue_idx(t + 2, (to + 2) % NI)

                if not skip_prev:
                    rpp, ipp = (to - 1) % NR, (to - 1) % NI
                    wait_rows(semg[rpp])
                    pltpu.async_copy(rows[rpp], acc_sh.at[idst[ipp]],
                                     sems[rpp], add=True)

            def final_scatter(to):
                rp, ip = to % NR, to % NI
                wait_rows(semg[rp])
                pltpu.async_copy(rows[rp], acc_sh.at[idst[ip]],
                                 sems[rp], add=True)

            issue_idx(0, 0)
            issue_idx(1, 1)
            do_block(0, 0, True, True)
            do_block(1, 1, True, False)
            do_block(2, 2, True, False)

            def main_body(it, carry):
                base = 3 + (NR * NI) * it
                for k in range(NR * NI):
                    do_block(base + k, 3 + k, False, False)
                return carry

            lax.fori_loop(0, main_iters, main_body, 0)

            for k in range(3):
                do_block(tail0 + k, tail0 + k, False, False)

            @pl.when(s < brem)
            def _():
                do_block(tail0 + 3, tail0 + 3, False, False)
                final_scatter(tail0 + 3)

            @pl.when(s >= brem)
            def _():
                final_scatter(tail0 + 2)

            for r in range(NR):
                wait_rows(sems[r])

        @pl.when(c == 0)
        def _():
            run_graph(hs_hbm, srcm_hbm, dstm_hbm)

        @pl.when(c == 1)
        def _():
            run_graph(hr_hbm, srcr_hbm, dstr_hbm)

        plsc.subcore_barrier()

        @pl.when(c == 0)
        def _():
            pltpu.sync_copy(acc_sh.at[pl.ds(s * rpt, rpt)],
                            aggs_hbm.at[pl.ds(s * rpt, rpt)])

            @pl.when(s == NS - 1)
            def _():
                pltpu.sync_copy(acc_sh.at[pl.ds(NS * rpt, tail)],
                                aggs_hbm.at[pl.ds(NS * rpt, tail)])

        @pl.when(c == 1)
        def _():
            pltpu.sync_copy(acc_sh.at[pl.ds(s * rpt, rpt)],
                            aggr_hbm.at[pl.ds(s * rpt, rpt)])

            @pl.when(s == NS - 1)
            def _():
                pltpu.sync_copy(acc_sh.at[pl.ds(NS * rpt, tail)],
                                aggr_hbm.at[pl.ds(NS * rpt, tail)])

    return scatter_kernel(hs, hr, srcm2, dstm2, srcr2, dstr2, zrows)


def _tc_mm1(x, sw1, rw1):
    n, d = x.shape
    g = n // ROW_BLK

    def body(x_ref, sw_ref, rw_ref, h1s_ref, h1r_ref):
        xs = x_ref[...]
        h1s_ref[...] = jnp.dot(xs, sw_ref[...],
                               preferred_element_type=jnp.float32)
        h1r_ref[...] = jnp.dot(xs, rw_ref[...],
                               preferred_element_type=jnp.float32)

    return pl.pallas_call(
        body,
        grid=(g,),
        in_specs=[
            pl.BlockSpec((ROW_BLK, d), lambda i: (i, 0)),
            pl.BlockSpec((d, d), lambda i: (0, 0)),
            pl.BlockSpec((d, d), lambda i: (0, 0)),
        ],
        out_specs=[
            pl.BlockSpec((ROW_BLK, d), lambda i: (i, 0)),
            pl.BlockSpec((ROW_BLK, d), lambda i: (i, 0)),
        ],
        out_shape=[
            jax.ShapeDtypeStruct((n, d), jnp.float32),
            jax.ShapeDtypeStruct((n, d), jnp.float32),
        ],
    )(x, sw1, rw1)


def _tc_scale(histp, h1s, h1r):
    n, d = h1s.shape
    g = n // ROW_BLK
    nt = histp.shape[0]

    def body(hp_ref, h1s_ref, h1r_ref, o1s_ref, o1r_ref, rsd_ref):
        deg = jnp.sum(hp_ref[...], axis=0)
        rsd = lax.rsqrt(jnp.maximum(deg, 1.0))
        rsd_ref[...] = rsd
        o1s_ref[...] = h1s_ref[...] * rsd[:, 0:1]
        o1r_ref[...] = h1r_ref[...] * rsd[:, 2:3]

    return pl.pallas_call(
        body,
        grid=(g,),
        in_specs=[
            pl.BlockSpec((nt, ROW_BLK, 4), lambda i: (0, i, 0)),
            pl.BlockSpec((ROW_BLK, d), lambda i: (i, 0)),
            pl.BlockSpec((ROW_BLK, d), lambda i: (i, 0)),
        ],
        out_specs=[
            pl.BlockSpec((ROW_BLK, d), lambda i: (i, 0)),
            pl.BlockSpec((ROW_BLK, d), lambda i: (i, 0)),
            pl.BlockSpec((ROW_BLK, 4), lambda i: (i, 0)),
        ],
        out_shape=[
            jax.ShapeDtypeStruct((n, d), jnp.float32),
            jax.ShapeDtypeStruct((n, d), jnp.float32),
            jax.ShapeDtypeStruct((n, 4), jnp.float32),
        ],
    )(histp, h1s, h1r)


def _elu(v):
    return jnp.where(v > 0, v, jnp.exp(jnp.minimum(v, 0.0)) - 1.0)


def _tc_stage_a(aggs, aggr, rsd, bs, br):
    n, d = aggs.shape
    g = n // ROW_BLK

    def body(as_ref, ar_ref, rsd_ref, bs_ref, br_ref, ys_ref, yr_ref, ps_ref):
        rsd = rsd_ref[...]
        ys = _elu(as_ref[...] * rsd[:, 1:2] + bs_ref[...][None, :])
        yr = _elu(ar_ref[...] * rsd[:, 3:4] + br_ref[...][None, :])
        ys_ref[...] = ys
        yr_ref[...] = yr
        ps_ref[...] = jnp.concatenate(
            [
                jnp.sum(ys, axis=0).reshape(1, 1, d),
                jnp.sum(ys * ys, axis=0).reshape(1, 1, d),
                jnp.sum(yr, axis=0).reshape(1, 1, d),
                jnp.sum(yr * yr, axis=0).reshape(1, 1, d),
            ],
            axis=1,
        )

    return pl.pallas_call(
        body,
        grid=(g,),
        in_specs=[
            pl.BlockSpec((ROW_BLK, d), lambda i: (i, 0)),
            pl.BlockSpec((ROW_BLK, d), lambda i: (i, 0)),
            pl.BlockSpec((ROW_BLK, 4), lambda i: (i, 0)),
            pl.BlockSpec((d,), lambda i: (0,)),
            pl.BlockSpec((d,), lambda i: (0,)),
        ],
        out_specs=[
            pl.BlockSpec((ROW_BLK, d), lambda i: (i, 0)),
            pl.BlockSpec((ROW_BLK, d), lambda i: (i, 0)),
            pl.BlockSpec((1, 4, d), lambda i: (i, 0, 0)),
        ],
        out_shape=[
            jax.ShapeDtypeStruct((n, d), jnp.float32),
            jax.ShapeDtypeStruct((n, d), jnp.float32),
            jax.ShapeDtypeStruct((g, 4, d), jnp.float32),
        ],
    )(aggs, aggr, rsd, bs, br)


def _bn_from_sums(y, tot_s1, tot_s2, n, gamma, beta):
    mu = tot_s1 / n
    var = tot_s2 / n - mu * mu
    return (y - mu) * lax.rsqrt(var + EPS) * gamma + beta


def _tc_mid_b(ys, yr, ps, rsd, gs, bes, gr, ber, sw2, rw2):
    n, d = ys.shape
    g = n // ROW_BLK

    def body(ys_ref, yr_ref, ps_ref, rsd_ref, gs_ref, bes_ref, gr_ref, ber_ref,
             sw_ref, rw_ref, h2s_ref, h2r_ref):
        tot = jnp.sum(ps_ref[...], axis=0)
        rsd = rsd_ref[...]
        zs = _bn_from_sums(ys_ref[...], tot[0:1], tot[1:2], n,
                           gs_ref[...][None, :], bes_ref[...][None, :])
        zr = _bn_from_sums(yr_ref[...], tot[2:3], tot[3:4], n,
                           gr_ref[...][None, :], ber_ref[...][None, :])
        h2s_ref[...] = jnp.dot(zs * rsd[:, 0:1], sw_ref[...],
                               preferred_element_type=jnp.float32)
        h2r_ref[...] = jnp.dot(zr * rsd[:, 2:3], rw_ref[...],
                               preferred_element_type=jnp.float32)

    return pl.pallas_call(
        body,
        grid=(g,),
        in_specs=[
            pl.BlockSpec((ROW_BLK, d), lambda i: (i, 0)),
            pl.BlockSpec((ROW_BLK, d), lambda i: (i, 0)),
            pl.BlockSpec((g, 4, d), lambda i: (0, 0, 0)),
            pl.BlockSpec((ROW_BLK, 4), lambda i: (i, 0)),
            pl.BlockSpec((d,), lambda i: (0,)),
            pl.BlockSpec((d,), lambda i: (0,)),
            pl.BlockSpec((d,), lambda i: (0,)),
            pl.BlockSpec((d,), lambda i: (0,)),
            pl.BlockSpec((d, d), lambda i: (0, 0)),
            pl.BlockSpec((d, d), lambda i: (0, 0)),
        ],
        out_specs=[
            pl.BlockSpec((ROW_BLK, d), lambda i: (i, 0)),
            pl.BlockSpec((ROW_BLK, d), lambda i: (i, 0)),
        ],
        out_shape=[
            jax.ShapeDtypeStruct((n, d), jnp.float32),
            jax.ShapeDtypeStruct((n, d), jnp.float32),
        ],
    )(ys, yr, ps, rsd, gs, bes, gr, ber, sw2, rw2)


def _tc_final(ys, yr, ps, gs, bes, gr, ber, gw1, gb1, pa, gw2, gb2, fw, fb):
    n, d = ys.shape
    c_out = fw.shape[1]
    g = n // ROW_BLK

    def body(ys_ref, yr_ref, ps_ref, gs_ref, bes_ref, gr_ref, ber_ref,
             gw1_ref, gb1_ref, pa_ref, gw2_ref, gb2_ref, fw_ref, fb_ref,
             out_ref):
        tot = jnp.sum(ps_ref[...], axis=0)
        h2 = _bn_from_sums(ys_ref[...], tot[0:1], tot[1:2], n,
                           gs_ref[...][None, :], bes_ref[...][None, :])
        h3 = _bn_from_sums(yr_ref[...], tot[2:3], tot[3:4], n,
                           gr_ref[...][None, :], ber_ref[...][None, :])
        diff = jnp.abs(h2 - h3)
        prod = h2 * h3
        z = (jnp.dot(h2, gw1_ref[0:d, :], preferred_element_type=jnp.float32)
             + jnp.dot(h3, gw1_ref[d:2 * d, :],
                       preferred_element_type=jnp.float32)
             + jnp.dot(diff, gw1_ref[2 * d:3 * d, :],
                       preferred_element_type=jnp.float32)
             + jnp.dot(prod, gw1_ref[3 * d:4 * d, :],
                       preferred_element_type=jnp.float32)
             + gb1_ref[...][None, :])
        z = jnp.where(z > 0, z, pa_ref[...] * z)
        zz = (jnp.dot(z, gw2_ref[...], preferred_element_type=jnp.float32)
              + gb2_ref[...][None, :])
        gate = 1.0 / (1.0 + jnp.exp(-zz))
        comb = gate * h2 + (1.0 - gate) * h3
        out_ref[...] = (jnp.dot(comb, fw_ref[...],
                                preferred_element_type=jnp.float32)
                        + fb_ref[...][None, :])

    return pl.pallas_call(
        body,
        grid=(g,),
        in_specs=[
            pl.BlockSpec((ROW_BLK, d), lambda i: (i, 0)),
            pl.BlockSpec((ROW_BLK, d), lambda i: (i, 0)),
            pl.BlockSpec((g, 4, d), lambda i: (0, 0, 0)),
            pl.BlockSpec((d,), lambda i: (0,)),
            pl.BlockSpec((d,), lambda i: (0,)),
            pl.BlockSpec((d,), lambda i: (0,)),
            pl.BlockSpec((d,), lambda i: (0,)),
            pl.BlockSpec((4 * d, d), lambda i: (0, 0)),
            pl.BlockSpec((d,), lambda i: (0,)),
            pl.BlockSpec((1, 1), lambda i: (0, 0)),
            pl.BlockSpec((d, d), lambda i: (0, 0)),
            pl.BlockSpec((d,), lambda i: (0,)),
            pl.BlockSpec((d, c_out), lambda i: (0, 0)),
            pl.BlockSpec((c_out,), lambda i: (0,)),
        ],
        out_specs=pl.BlockSpec((ROW_BLK, c_out), lambda i: (i, 0)),
        out_shape=jax.ShapeDtypeStruct((n, c_out), jnp.float32),
    )(ys, yr, ps, gs, bes, gr, ber, gw1, gb1, pa, gw2, gb2, fw, fb)


def kernel(node_features, mention_edges, retweet_edges, sW1, sb1, sg1, sbe1,
           sW2, sb2, sg2, sbe2, rW1, rb1, rg1, rbe1, rW2, rb2, rg2, rbe2,
           gW1, gb1, pa, gW2, gb2, fW, fb):
    n, d = node_features.shape
    srcm, dstm = mention_edges[0], mention_edges[1]
    srcr, dstr = retweet_edges[0], retweet_edges[1]
    srcm2, dstm2 = srcm.reshape(-1, 128), dstm.reshape(-1, 128)
    srcr2, dstr2 = srcr.reshape(-1, 128), dstr.reshape(-1, 128)

    zrows = jnp.zeros((128, d), jnp.float32)
    hrp = ((4 * n + 127) // 128 + 127) // 128 * 128
    iota3 = jnp.arange(hrp, dtype=jnp.int32).reshape(hrp // 128, 128)
    histp = _sc_hist(srcm, dstm, srcr, dstr, iota3, zrows, n)
    histp = histp.reshape(NC, hrp * 128)[:, :4 * n].reshape(NC, n, 4)
    h1s0, h1r0 = _tc_mm1(node_features, sW1, rW1)
    h1s, h1r, rsd = _tc_scale(histp, h1s0, h1r0)
    agg1s, agg1r = _sc_scatter(h1s, h1r, srcm2, dstm2, srcr2, dstr2, zrows)
    ys1, yr1, ps1 = _tc_stage_a(agg1s, agg1r, rsd, sb1, rb1)
    h2s, h2r = _tc_mid_b(ys1, yr1, ps1, rsd, sg1, sbe1, rg1, rbe1, sW2, rW2)

    agg2s, agg2r = _sc_scatter(h2s, h2r, srcm2, dstm2, srcr2, dstr2, zrows)
    ys2, yr2, ps2 = _tc_stage_a(agg2s, agg2r, rsd, sb2, rb2)

    return _tc_final(ys2, yr2, ps2, sg2, sbe2, rg2, rbe2,
                     gW1, gb1, pa.reshape(1, 1), gW2, gb2, fW, fb)
